# Initial kernel scaffold; baseline (speedup 1.0000x reference)
import functools

import jax
import jax.numpy as jnp
from jax import lax
from jax.experimental import pallas as pl
from jax.experimental.pallas import tpu as pltpu
from jax.experimental.pallas import tpu_sc as plsc

N = 10000
NP = 10240
E = 320000
G = 256
H = 128
ED = 16

BLK = 1024
EB = 2560

NC = 2
NS = 16
NW = NC * NS
EPW = E // NW
K = 80
CH = EPW // K
ZR = NP // NS

_f32 = jnp.float32


def _leaky(x):
    return jnp.maximum(x, 0.01 * x)


def _elu(x):
    return jnp.where(x > 0, x, jnp.exp(jnp.minimum(x, 0.0)) - 1.0)


def _gru_math(xin, hprev, WihT, WhhT, bih, bhh):
    gi = jnp.dot(xin, WihT, preferred_element_type=_f32) + bih
    gh = jnp.dot(hprev, WhhT, preferred_element_type=_f32) + bhh
    r = jax.nn.sigmoid(gi[:, :H] + gh[:, :H])
    z = jax.nn.sigmoid(gi[:, H:2 * H] + gh[:, H:2 * H])
    n = jnp.tanh(gi[:, 2 * H:] + r * gh[:, 2 * H:])
    return (1.0 - z) * n + z * hprev


def _tc1_body(x_ref, weT, be, wlT, bl, a1T, g2T, ar,
              xh_ref, p_ref, s_ref, r_ref):
    xh0 = jnp.dot(x_ref[...], weT[...], preferred_element_type=_f32) + be[...]
    xh = _leaky(jnp.dot(xh0, wlT[...], preferred_element_type=_f32) + bl[...])
    xh_ref[...] = xh
    p_ref[...] = jnp.dot(xh, a1T[...], preferred_element_type=_f32)
    s_ref[...] = jnp.dot(xh, g2T[...], preferred_element_type=_f32)
    r_ref[...] = jnp.sum(xh * ar[...], axis=1, keepdims=True)


def _tc1(xp, weT, be, wlT, bl, a1T, g2T, ar):
    full = lambda s: pl.BlockSpec(s, lambda i: (0, 0))
    nblk = lambda s: pl.BlockSpec(s, lambda i: (i, 0))
    return pl.pallas_call(
        _tc1_body,
        grid=(NP // BLK,),
        in_specs=[nblk((BLK, H)), full((H, H)), full((1, H)), full((H, H)),
                  full((1, H)), full((H, H)), full((H, H)), full((1, H))],
        out_specs=[nblk((BLK, H)), nblk((BLK, H)), nblk((BLK, H)),
                   nblk((BLK, 1))],
        out_shape=[jax.ShapeDtypeStruct((NP, H), _f32),
                   jax.ShapeDtypeStruct((NP, H), _f32),
                   jax.ShapeDtypeStruct((NP, H), _f32),
                   jax.ShapeDtypeStruct((NP, 1), _f32)],
    )(xp, weT, be, wlT, bl, a1T, g2T, ar)


def _tc2_body(pj_ref, ea_ref, b2T, al, t_ref):
    z = pj_ref[...] + jnp.dot(ea_ref[...], b2T[...],
                              preferred_element_type=_f32)
    m = _leaky(z)
    t_ref[...] = jnp.sum(m * al[...], axis=1, keepdims=True)


def _tc2(pj, ea, b2T, al):
    full = lambda s: pl.BlockSpec(s, lambda i: (0, 0))
    eblk = lambda s: pl.BlockSpec(s, lambda i: (i, 0))
    return pl.pallas_call(
        _tc2_body,
        grid=(E // EB,),
        in_specs=[eblk((EB, H)), eblk((EB, ED)), full((ED, H)), full((1, H))],
        out_specs=[eblk((EB, 1))],
        out_shape=[jax.ShapeDtypeStruct((E, 1), _f32)],
    )(pj, ea, b2T, al)[0]


def _tc3_body(accr_ref, accw_ref, xh_ref, gcb, wihT, whhT, bih, bhh,
              aclT, asrc, adst,
              xh2_ref, xl_ref, u_ref, v_ref):
    hsum = accr_ref[0] + accr_ref[1]
    den = accw_ref[0, :, 0:1] + accw_ref[1, :, 0:1]
    h = _elu(hsum / (den + 1e-16) + gcb[...])
    xh2 = jnp.maximum(
        _gru_math(h, xh_ref[...], wihT[...], whhT[...], bih[...], bhh[...]),
        0.0)
    xh2_ref[...] = xh2
    xl = jnp.dot(xh2, aclT[...], preferred_element_type=_f32)
    xl_ref[...] = xl
    u_ref[...] = jnp.sum(xl * asrc[...], axis=1, keepdims=True)
    v_ref[...] = jnp.sum(xl * adst[...], axis=1, keepdims=True)


def _tc3(accr, accw, xh, gcb, wihT, whhT, bih, bhh, aclT, asrc, adst):
    full = lambda s: pl.BlockSpec(s, lambda i: (0, 0))
    nblk = lambda s: pl.BlockSpec(s, lambda i: (i, 0))
    return pl.pallas_call(
        _tc3_body,
        grid=(NP // BLK,),
        in_specs=[pl.BlockSpec((NC, BLK, H), lambda i: (0, i, 0)),
                  pl.BlockSpec((NC, BLK, 16), lambda i: (0, i, 0)),
                  nblk((BLK, H)), full((1, H)),
                  full((H, 3 * H)), full((H, 3 * H)),
                  full((1, 3 * H)), full((1, 3 * H)),
                  full((H, H)), full((1, H)), full((1, H))],
        out_specs=[nblk((BLK, H)), nblk((BLK, H)), nblk((BLK, 1)),
                   nblk((BLK, 1))],
        out_shape=[jax.ShapeDtypeStruct((NP, H), _f32),
                   jax.ShapeDtypeStruct((NP, H), _f32),
                   jax.ShapeDtypeStruct((NP, 1), _f32),
                   jax.ShapeDtypeStruct((NP, 1), _f32)],
    )(accr, accw, xh, gcb, wihT, whhT, bih, bhh, aclT, asrc, adst)


def _tc4a_body(accr_ref, accw_ref, xh2_ref, acb, wihT, whhT, bih, bhh,
               mclT, msrc, bat_ref,
               xs_ref, su_ref, seg_ref):
    i = pl.program_id(0)
    ng = pl.num_programs(0)
    hsum = accr_ref[0] + accr_ref[1]
    den = accw_ref[0, :, 0:1] + accw_ref[1, :, 0:1]
    h2 = _elu(hsum / (den + 1e-16) + acb[...])
    xh3 = jnp.maximum(
        _gru_math(h2, xh2_ref[...], wihT[...], whhT[...], bih[...], bhh[...]),
        0.0)
    xs = jnp.dot(xh3, mclT[...], preferred_element_type=_f32)
    xs_ref[...] = xs
    su_ref[...] = jnp.sum(xs * msrc[...], axis=1, keepdims=True)
    gidx = lax.broadcasted_iota(jnp.int32, (BLK, G), 1)
    mask = (bat_ref[...] == gidx).astype(_f32)
    contrib = lax.dot_general(mask, xh3, (((0,), (0,)), ((), ())),
                              preferred_element_type=_f32)

    @pl.when(i == 0)
    def _():
        seg_ref[...] = contrib

    @pl.when(i > 0)
    def _():
        seg_ref[...] = seg_ref[...] + contrib

    @pl.when(i == ng - 1)
    def _():
        seg_ref[...] = jnp.maximum(seg_ref[...], 0.0)


def _tc4a(accr, accw, xh2, acb, wihT, whhT, bih, bhh, mclT, msrc, batp):
    full = lambda s: pl.BlockSpec(s, lambda i: (0, 0))
    nblk = lambda s: pl.BlockSpec(s, lambda i: (i, 0))
    return pl.pallas_call(
        _tc4a_body,
        grid=(NP // BLK,),
        in_specs=[pl.BlockSpec((NC, BLK, H), lambda i: (0, i, 0)),
                  pl.BlockSpec((NC, BLK, 16), lambda i: (0, i, 0)),
                  nblk((BLK, H)), full((1, H)),
                  full((H, 3 * H)), full((H, 3 * H)),
                  full((1, 3 * H)), full((1, 3 * H)),
                  full((H, H)), full((1, H)), nblk((BLK, 1))],
        out_specs=[nblk((BLK, H)), nblk((BLK, 1)), full((G, H))],
        out_shape=[jax.ShapeDtypeStruct((NP, H), _f32),
                   jax.ShapeDtypeStruct((NP, 1), _f32),
                   jax.ShapeDtypeStruct((G, H), _f32)],
    )(accr, accw, xh2, acb, wihT, whhT, bih, bhh, mclT, msrc, batp)


def _tc4b_body(out_ref, xs_ref, su_ref, bat_ref, mclT, mdst, mcb,
               wihT, whhT, bih, bhh,
               outnew_ref, num_s, den_s):
    i = pl.program_id(0)
    ng = pl.num_programs(0)
    outv = out_ref[...]
    od = jnp.dot(outv, mclT[...], preferred_element_type=_f32)
    sv = jnp.sum(od * mdst[...], axis=1, keepdims=True)
    gidx = lax.broadcasted_iota(jnp.int32, (BLK, G), 1)
    mask = (bat_ref[...] == gidx).astype(_f32)
    svn = lax.dot_general(mask, sv, (((1,), (0,)), ((), ())),
                          preferred_element_type=_f32)
    w = jnp.exp(_leaky(su_ref[...] + svn))
    nc = lax.dot_general(mask, w * xs_ref[...], (((0,), (0,)), ((), ())),
                         preferred_element_type=_f32)
    dc = lax.dot_general(mask, jnp.broadcast_to(w, (BLK, H)),
                         (((0,), (0,)), ((), ())),
                         preferred_element_type=_f32)

    @pl.when(i == 0)
    def _():
        num_s[...] = nc
        den_s[...] = dc

    @pl.when(i > 0)
    def _():
        num_s[...] = num_s[...] + nc
        den_s[...] = den_s[...] + dc

    @pl.when(i == ng - 1)
    def _():
        hm = _elu(num_s[...] / (den_s[...] + 1e-16) + mcb[...])
        outnew_ref[...] = jnp.maximum(
            _gru_math(hm, outv, wihT[...], whhT[...], bih[...], bhh[...]),
            0.0)


def _tc4b(out, xs, su, batp, mclT, mdst, mcb, wihT, whhT, bih, bhh):
    full = lambda s: pl.BlockSpec(s, lambda i: (0, 0))
    nblk = lambda s: pl.BlockSpec(s, lambda i: (i, 0))
    return pl.pallas_call(
        _tc4b_body,
        grid=(NP // BLK,),
        in_specs=[full((G, H)), nblk((BLK, H)), nblk((BLK, 1)),
                  nblk((BLK, 1)), full((H, H)), full((1, H)), full((1, H)),
                  full((H, 3 * H)), full((H, 3 * H)),
                  full((1, 3 * H)), full((1, 3 * H))],
        out_specs=[full((G, H))],
        out_shape=[jax.ShapeDtypeStruct((G, H), _f32)],
        scratch_shapes=[pltpu.VMEM((G, H), _f32), pltpu.VMEM((G, H), _f32)],
    )(out, xs, su, batp, mclT, mdst, mcb, wihT, whhT, bih, bhh)[0]


def _tc4d_body(out_ref, wl2T, bl2, wt1T, bt1, wt2T, bt2, y_ref):
    fp = jnp.dot(out_ref[...], wl2T[...], preferred_element_type=_f32) \
        + bl2[...]
    hh = jnp.maximum(
        jnp.dot(fp, wt1T[...], preferred_element_type=_f32) + bt1[...], 0.0)
    y_ref[...] = jnp.dot(hh, wt2T[...], preferred_element_type=_f32) \
        + bt2[...]


def _tc4d(out, wl2T, bl2, wt1T, bt1, wt2T, bt2):
    full = lambda s: pl.BlockSpec(s, lambda: (0, 0))
    return pl.pallas_call(
        _tc4d_body,
        in_specs=[full((G, H)), full((H, H)), full((1, H)),
                  full((H, 64)), full((1, 64)), full((64, H)), full((1, H))],
        out_specs=full((G, H)),
        out_shape=jax.ShapeDtypeStruct((G, H), _f32),
    )(out, wl2T, bl2, wt1T, bt1, wt2T, bt2)


def _sc_mesh():
    return plsc.VectorSubcoreMesh(core_axis_name="c", subcore_axis_name="s")


def _sc_gather(table, idx):
    @functools.partial(
        pl.kernel,
        out_type=jax.ShapeDtypeStruct((E, H), _f32),
        mesh=_sc_mesh(),
        scratch_types=[pltpu.VMEM((K,), jnp.int32),
                       pltpu.VMEM((K, H), _f32),
                       pltpu.SemaphoreType.DMA],
    )
    def k(tab_hbm, idx_hbm, out_hbm, idxv, rows, sem):
        cid = lax.axis_index("c")
        sid = lax.axis_index("s")
        base = (cid * NS + sid) * EPW

        @pl.loop(0, CH)
        def _(c):
            off = base + c * K
            pltpu.sync_copy(idx_hbm.at[pl.ds(off, K)], idxv)
            pltpu.async_copy(tab_hbm.at[idxv], rows, sem).wait()
            pltpu.sync_copy(rows, out_hbm.at[pl.ds(off, K)])

    return k(table, idx)


def _sc_aggregate(table, tscal, bscal, cscal, src, dst):
    @functools.partial(
        pl.kernel,
        out_type=(jax.ShapeDtypeStruct((NC, NP, H), _f32),
                  jax.ShapeDtypeStruct((NC, NP, 16), _f32)),
        mesh=_sc_mesh(),
        scratch_types=[pltpu.VMEM((NP,), _f32),
                       pltpu.VMEM((NP,), _f32),
                       pltpu.VMEM((K,), jnp.int32),
                       pltpu.VMEM((K,), jnp.int32),
                       pltpu.VMEM((K,), _f32),
                       pltpu.VMEM((K, H), _f32),
                       pltpu.VMEM((K, 16), _f32),
                       pltpu.VMEM((K,), _f32),
                       pltpu.VMEM_SHARED((NP, H), _f32),
                       pltpu.VMEM_SHARED((NP, 16), _f32),
                       pltpu.SemaphoreType.DMA],
    )
    def k(tab_hbm, t_hbm, b_hbm, c_hbm, src_hbm, dst_hbm,
          accr_hbm, accw_hbm,
          bloc, cloc, srcv, dstv, tv, rows, wcol, wv, accr_sh, accw_sh, sem):
        cid = lax.axis_index("c")
        sid = lax.axis_index("s")
        zv = jnp.zeros((16,), _f32)

        @pl.loop(0, K)
        def _(i):
            for j in range(H // 16):
                rows[i, pl.ds(j * 16, 16)] = zv
            wcol[i, :] = zv

        @pl.loop(0, ZR // K)
        def _(ci):
            b0 = sid * ZR + ci * K
            pltpu.sync_copy(rows, accr_sh.at[pl.ds(b0, K)])
            pltpu.sync_copy(wcol, accw_sh.at[pl.ds(b0, K)])

        pltpu.sync_copy(b_hbm, bloc)
        pltpu.sync_copy(c_hbm, cloc)
        plsc.subcore_barrier()

        base = (cid * NS + sid) * EPW
        col0 = jnp.zeros((16,), jnp.int32)

        @pl.loop(0, CH)
        def _(c):
            off = base + c * K
            pltpu.sync_copy(src_hbm.at[pl.ds(off, K)], srcv)
            pltpu.sync_copy(dst_hbm.at[pl.ds(off, K)], dstv)
            pltpu.sync_copy(t_hbm.at[pl.ds(off, K)], tv)
            pltpu.async_copy(tab_hbm.at[srcv], rows, sem).wait()

            @pl.loop(0, K // 16)
            def _(g):
                sl = pl.ds(g * 16, 16)
                bv = plsc.load_gather(bloc, [srcv[sl]])
                cv = plsc.load_gather(cloc, [dstv[sl]])
                gs = tv[sl] + bv + cv
                w = jnp.exp(jnp.maximum(gs, 0.01 * gs))
                wv[sl] = w
                ridx = lax.iota(jnp.int32, 16) + g * 16
                plsc.store_scatter(wcol, [ridx, col0], w)

            @pl.loop(0, K)
            def _(i):
                ws = wv[i]
                for j in range(H // 16):
                    sl = pl.ds(j * 16, 16)
                    rows[i, sl] = rows[i, sl] * ws

            pltpu.sync_copy(rows, accr_sh.at[dstv], add=True)
            pltpu.sync_copy(wcol, accw_sh.at[dstv], add=True)

        plsc.subcore_barrier()

        @pl.loop(0, ZR // K)
        def _(ci):
            b0 = sid * ZR + ci * K
            pltpu.sync_copy(accr_sh.at[pl.ds(b0, K)],
                            accr_hbm.at[cid, pl.ds(b0, K)])
            pltpu.sync_copy(accw_sh.at[pl.ds(b0, K)],
                            accw_hbm.at[cid, pl.ds(b0, K)])

    return k(table, tscal, bscal, cscal, src, dst)


def kernel(x, edge_index, edge_attr, batch, W_embed, b_embed, W_lin1, b_lin1,
           gc_lin1, gc_lin2, gc_att_l, gc_att_r, gc_bias,
           gru1_Wih, gru1_Whh, gru1_bih, gru1_bhh,
           ac_lin, ac_att_src, ac_att_dst, ac_bias,
           gru2_Wih, gru2_Whh, gru2_bih, gru2_bhh,
           mc_lin, mc_att_src, mc_att_dst, mc_bias,
           mgru_Wih, mgru_Whh, mgru_bih, mgru_bhh,
           W_lin2, b_lin2, W_t1, b_t1, W_t2, b_t2):
    src = edge_index[0]
    dst = edge_index[1]
    xp = jnp.pad(x, ((0, NP - N), (0, 0)))
    batp = jnp.pad(batch, (0, NP - N), constant_values=G).reshape(NP, 1)

    row = lambda b: b.reshape(1, -1)

    xh, p, s, r1 = _tc1(xp, W_embed.T, row(b_embed), W_lin1.T, row(b_lin1),
                        gc_lin1[:, :H].T, gc_lin2.T, row(gc_att_r))

    pj = _sc_gather(p, src)

    t1 = _tc2(pj, edge_attr, gc_lin1[:, H:].T, row(gc_att_l))

    zeros_n = jnp.zeros((NP,), _f32)
    accr, accw = _sc_aggregate(s, t1.reshape(E), zeros_n, r1.reshape(NP),
                               src, dst)

    xh2, xl, u, v = _tc3(accr, accw, xh, row(gc_bias),
                         gru1_Wih.T, gru1_Whh.T, row(gru1_bih), row(gru1_bhh),
                         ac_lin.T, row(ac_att_src), row(ac_att_dst))

    zeros_e = jnp.zeros((E,), _f32)
    acc2r, acc2w = _sc_aggregate(xl, zeros_e, u.reshape(NP), v.reshape(NP),
                                 src, dst)

    xs, su, out0 = _tc4a(acc2r, acc2w, xh2, row(ac_bias),
                         gru2_Wih.T, gru2_Whh.T, row(gru2_bih), row(gru2_bhh),
                         mc_lin.T, row(mc_att_src), batp)

    out = out0
    for _ in range(2):
        out = _tc4b(out, xs, su, batp, mc_lin.T, row(mc_att_dst),
                    row(mc_bias), mgru_Wih.T, mgru_Whh.T, row(mgru_bih),
                    row(mgru_bhh))

    wt2T = jnp.zeros((64, H), _f32).at[:, 0].set(W_t2[0])
    bt2 = jnp.zeros((1, H), _f32).at[0, 0].set(b_t2[0])
    yfull = _tc4d(out, W_lin2.T, row(b_lin2), W_t1.T, row(b_t1), wt2T, bt2)
    return yfull[:, 0:1]


# speedup vs baseline: 11.1191x; 11.1191x over previous
import functools

import jax
import jax.numpy as jnp
from jax import lax
from jax.experimental import pallas as pl
from jax.experimental.pallas import tpu as pltpu
from jax.experimental.pallas import tpu_sc as plsc

N = 10000
NP = 10240
E = 320000
G = 256
H = 128
HE = 144
ED = 16

BLK = 1024
EB = 2560

NC = 2
NS = 16
NW = NC * NS
EPW = E // NW
K = 80
CH = EPW // K
ZR = NP // NS

_f32 = jnp.float32


def _leaky(x):
    return jnp.maximum(x, 0.01 * x)


def _elu(x):
    return jnp.where(x > 0, x, jnp.exp(jnp.minimum(x, 0.0)) - 1.0)


def _gru_math(xin, hprev, WihT, WhhT, bih, bhh):
    gi = jnp.dot(xin, WihT, preferred_element_type=_f32) + bih
    gh = jnp.dot(hprev, WhhT, preferred_element_type=_f32) + bhh
    r = jax.nn.sigmoid(gi[:, :H] + gh[:, :H])
    z = jax.nn.sigmoid(gi[:, H:2 * H] + gh[:, H:2 * H])
    n = jnp.tanh(gi[:, 2 * H:] + r * gh[:, 2 * H:])
    return (1.0 - z) * n + z * hprev


def _ext16(u):
    li = lax.broadcasted_iota(jnp.int32, (BLK, 16), 1)
    return jnp.where(li == 0, u, 0.0) + jnp.where(li == 1, 1.0, 0.0)


def _tc1_body(x_ref, weT, be, wlT, bl, a1T, g2T, ar,
              xh_ref, p_ref, s_ref, r_ref):
    xh0 = jnp.dot(x_ref[...], weT[...], preferred_element_type=_f32) + be[...]
    xh = _leaky(jnp.dot(xh0, wlT[...], preferred_element_type=_f32) + bl[...])
    xh_ref[...] = xh
    p_ref[...] = jnp.dot(xh, a1T[...], preferred_element_type=_f32)
    s_ref[:, :H] = jnp.dot(xh, g2T[...], preferred_element_type=_f32)
    s_ref[:, H:] = _ext16(jnp.zeros((BLK, 1), _f32))
    r_ref[...] = jnp.sum(xh * ar[...], axis=1, keepdims=True)


def _tc1(xp, weT, be, wlT, bl, a1T, g2T, ar):
    full = lambda s: pl.BlockSpec(s, lambda i: (0, 0))
    nblk = lambda s: pl.BlockSpec(s, lambda i: (i, 0))
    return pl.pallas_call(
        _tc1_body,
        grid=(NP // BLK,),
        in_specs=[nblk((BLK, H)), full((H, H)), full((1, H)), full((H, H)),
                  full((1, H)), full((H, H)), full((H, H)), full((1, H))],
        out_specs=[nblk((BLK, H)), nblk((BLK, H)), nblk((BLK, HE)),
                   nblk((BLK, 1))],
        out_shape=[jax.ShapeDtypeStruct((NP, H), _f32),
                   jax.ShapeDtypeStruct((NP, H), _f32),
                   jax.ShapeDtypeStruct((NP, HE), _f32),
                   jax.ShapeDtypeStruct((NP, 1), _f32)],
    )(xp, weT, be, wlT, bl, a1T, g2T, ar)


def _tc2_body(pj_ref, ea_ref, b2T, al, t_ref):
    z = pj_ref[...] + jnp.dot(ea_ref[...], b2T[...],
                              preferred_element_type=_f32)
    m = _leaky(z)
    t_ref[...] = jnp.sum(m * al[...], axis=1, keepdims=True)


def _tc2(pj, ea, b2T, al):
    full = lambda s: pl.BlockSpec(s, lambda i: (0, 0))
    eblk = lambda s: pl.BlockSpec(s, lambda i: (i, 0))
    return pl.pallas_call(
        _tc2_body,
        grid=(E // EB,),
        in_specs=[eblk((EB, H)), eblk((EB, ED)), full((ED, H)), full((1, H))],
        out_specs=[eblk((EB, 1))],
        out_shape=[jax.ShapeDtypeStruct((E, 1), _f32)],
    )(pj, ea, b2T, al)[0]


def _tc3_body(acc_ref, xh_ref, gcb, wihT, whhT, bih, bhh,
              aclT, asrc, adst,
              xh2_ref, xl_ref, v_ref):
    hsum = acc_ref[0, :, :H] + acc_ref[1, :, :H]
    den = acc_ref[0, :, H + 1:H + 2] + acc_ref[1, :, H + 1:H + 2]
    h = _elu(hsum / (den + 1e-16) + gcb[...])
    xh2 = jnp.maximum(
        _gru_math(h, xh_ref[...], wihT[...], whhT[...], bih[...], bhh[...]),
        0.0)
    xh2_ref[...] = xh2
    xl = jnp.dot(xh2, aclT[...], preferred_element_type=_f32)
    xl_ref[:, :H] = xl
    xl_ref[:, H:] = _ext16(jnp.sum(xl * asrc[...], axis=1, keepdims=True))
    v_ref[...] = jnp.sum(xl * adst[...], axis=1, keepdims=True)


def _tc3(acc, xh, gcb, wihT, whhT, bih, bhh, aclT, asrc, adst):
    full = lambda s: pl.BlockSpec(s, lambda i: (0, 0))
    nblk = lambda s: pl.BlockSpec(s, lambda i: (i, 0))
    return pl.pallas_call(
        _tc3_body,
        grid=(NP // BLK,),
        in_specs=[pl.BlockSpec((NC, BLK, HE), lambda i: (0, i, 0)),
                  nblk((BLK, H)), full((1, H)),
                  full((H, 3 * H)), full((H, 3 * H)),
                  full((1, 3 * H)), full((1, 3 * H)),
                  full((H, H)), full((1, H)), full((1, H))],
        out_specs=[nblk((BLK, H)), nblk((BLK, HE)), nblk((BLK, 1))],
        out_shape=[jax.ShapeDtypeStruct((NP, H), _f32),
                   jax.ShapeDtypeStruct((NP, HE), _f32),
                   jax.ShapeDtypeStruct((NP, 1), _f32)],
    )(acc, xh, gcb, wihT, whhT, bih, bhh, aclT, asrc, adst)


def _tc4a_body(acc_ref, xh2_ref, acb, wihT, whhT, bih, bhh,
               mclT, msrc, bat_ref,
               xs_ref, su_ref, seg_ref):
    i = pl.program_id(0)
    ng = pl.num_programs(0)
    hsum = acc_ref[0, :, :H] + acc_ref[1, :, :H]
    den = acc_ref[0, :, H + 1:H + 2] + acc_ref[1, :, H + 1:H + 2]
    h2 = _elu(hsum / (den + 1e-16) + acb[...])
    xh3 = jnp.maximum(
        _gru_math(h2, xh2_ref[...], wihT[...], whhT[...], bih[...], bhh[...]),
        0.0)
    xs = jnp.dot(xh3, mclT[...], preferred_element_type=_f32)
    xs_ref[...] = xs
    su_ref[...] = jnp.sum(xs * msrc[...], axis=1, keepdims=True)
    gidx = lax.broadcasted_iota(jnp.int32, (BLK, G), 1)
    mask = (bat_ref[...] == gidx).astype(_f32)
    contrib = lax.dot_general(mask, xh3, (((0,), (0,)), ((), ())),
                              preferred_element_type=_f32)

    @pl.when(i == 0)
    def _():
        seg_ref[...] = contrib

    @pl.when(i > 0)
    def _():
        seg_ref[...] = seg_ref[...] + contrib

    @pl.when(i == ng - 1)
    def _():
        seg_ref[...] = jnp.maximum(seg_ref[...], 0.0)


def _tc4a(acc, xh2, acb, wihT, whhT, bih, bhh, mclT, msrc, batp):
    full = lambda s: pl.BlockSpec(s, lambda i: (0, 0))
    nblk = lambda s: pl.BlockSpec(s, lambda i: (i, 0))
    return pl.pallas_call(
        _tc4a_body,
        grid=(NP // BLK,),
        in_specs=[pl.BlockSpec((NC, BLK, HE), lambda i: (0, i, 0)),
                  nblk((BLK, H)), full((1, H)),
                  full((H, 3 * H)), full((H, 3 * H)),
                  full((1, 3 * H)), full((1, 3 * H)),
                  full((H, H)), full((1, H)), nblk((BLK, 1))],
        out_specs=[nblk((BLK, H)), nblk((BLK, 1)), full((G, H))],
        out_shape=[jax.ShapeDtypeStruct((NP, H), _f32),
                   jax.ShapeDtypeStruct((NP, 1), _f32),
                   jax.ShapeDtypeStruct((G, H), _f32)],
    )(acc, xh2, acb, wihT, whhT, bih, bhh, mclT, msrc, batp)


def _tc4b_body(out_ref, xs_ref, su_ref, bat_ref, mclT, mdst, mcb,
               wihT, whhT, bih, bhh,
               outnew_ref, num_s, den_s):
    i = pl.program_id(0)
    ng = pl.num_programs(0)
    outv = out_ref[...]
    od = jnp.dot(outv, mclT[...], preferred_element_type=_f32)
    sv = jnp.sum(od * mdst[...], axis=1, keepdims=True)
    gidx = lax.broadcasted_iota(jnp.int32, (BLK, G), 1)
    mask = (bat_ref[...] == gidx).astype(_f32)
    svn = lax.dot_general(mask, sv, (((1,), (0,)), ((), ())),
                          preferred_element_type=_f32)
    w = jnp.exp(_leaky(su_ref[...] + svn))
    nc = lax.dot_general(mask, w * xs_ref[...], (((0,), (0,)), ((), ())),
                         preferred_element_type=_f32)
    dc = lax.dot_general(mask, jnp.broadcast_to(w, (BLK, H)),
                         (((0,), (0,)), ((), ())),
                         preferred_element_type=_f32)

    @pl.when(i == 0)
    def _():
        num_s[...] = nc
        den_s[...] = dc

    @pl.when(i > 0)
    def _():
        num_s[...] = num_s[...] + nc
        den_s[...] = den_s[...] + dc

    @pl.when(i == ng - 1)
    def _():
        hm = _elu(num_s[...] / (den_s[...] + 1e-16) + mcb[...])
        outnew_ref[...] = jnp.maximum(
            _gru_math(hm, outv, wihT[...], whhT[...], bih[...], bhh[...]),
            0.0)


def _tc4b(out, xs, su, batp, mclT, mdst, mcb, wihT, whhT, bih, bhh):
    full = lambda s: pl.BlockSpec(s, lambda i: (0, 0))
    nblk = lambda s: pl.BlockSpec(s, lambda i: (i, 0))
    return pl.pallas_call(
        _tc4b_body,
        grid=(NP // BLK,),
        in_specs=[full((G, H)), nblk((BLK, H)), nblk((BLK, 1)),
                  nblk((BLK, 1)), full((H, H)), full((1, H)), full((1, H)),
                  full((H, 3 * H)), full((H, 3 * H)),
                  full((1, 3 * H)), full((1, 3 * H))],
        out_specs=[full((G, H))],
        out_shape=[jax.ShapeDtypeStruct((G, H), _f32)],
        scratch_shapes=[pltpu.VMEM((G, H), _f32), pltpu.VMEM((G, H), _f32)],
    )(out, xs, su, batp, mclT, mdst, mcb, wihT, whhT, bih, bhh)[0]


def _tc4d_body(out_ref, wl2T, bl2, wt1T, bt1, wt2T, bt2, y_ref):
    fp = jnp.dot(out_ref[...], wl2T[...], preferred_element_type=_f32) \
        + bl2[...]
    hh = jnp.maximum(
        jnp.dot(fp, wt1T[...], preferred_element_type=_f32) + bt1[...], 0.0)
    y_ref[...] = jnp.dot(hh, wt2T[...], preferred_element_type=_f32) \
        + bt2[...]


def _tc4d(out, wl2T, bl2, wt1T, bt1, wt2T, bt2):
    full = lambda s: pl.BlockSpec(s, lambda: (0, 0))
    return pl.pallas_call(
        _tc4d_body,
        in_specs=[full((G, H)), full((H, H)), full((1, H)),
                  full((H, 64)), full((1, 64)), full((64, H)), full((1, H))],
        out_specs=full((G, H)),
        out_shape=jax.ShapeDtypeStruct((G, H), _f32),
    )(out, wl2T, bl2, wt1T, bt1, wt2T, bt2)


def _sc_mesh():
    return plsc.VectorSubcoreMesh(core_axis_name="c", subcore_axis_name="s")


_SC_PARAMS = pltpu.CompilerParams(needs_layout_passes=False,
                                  use_tc_tiling_on_sc=False)


def _sc_gather(table, idx):
    @functools.partial(
        pl.kernel,
        out_type=jax.ShapeDtypeStruct((E, H), _f32),
        mesh=_sc_mesh(),
        compiler_params=_SC_PARAMS,
        scratch_types=[pltpu.VMEM((K,), jnp.int32),
                       pltpu.VMEM((K, H), _f32),
                       pltpu.SemaphoreType.DMA],
    )
    def k(tab_hbm, idx_hbm, out_hbm, idxv, rows, sem):
        cid = lax.axis_index("c")
        sid = lax.axis_index("s")
        base = (cid * NS + sid) * EPW

        @pl.loop(0, CH)
        def _(c):
            off = base + c * K
            pltpu.sync_copy(idx_hbm.at[pl.ds(off, K)], idxv)
            pltpu.async_copy(tab_hbm.at[idxv], rows, sem).wait()
            pltpu.sync_copy(rows, out_hbm.at[pl.ds(off, K)])

    return k(table, idx)


def _sc_aggregate(table_ext, tscal, cscal, src, dst):
    @functools.partial(
        pl.kernel,
        out_type=jax.ShapeDtypeStruct((NC, NP, HE), _f32),
        mesh=_sc_mesh(),
        compiler_params=_SC_PARAMS,
        scratch_types=[pltpu.VMEM((NP,), _f32),
                       pltpu.VMEM((K,), jnp.int32),
                       pltpu.VMEM((K,), jnp.int32),
                       pltpu.VMEM((K,), _f32),
                       pltpu.VMEM((K, HE), _f32),
                       pltpu.VMEM_SHARED((NP, HE), _f32),
                       pltpu.SemaphoreType.DMA],
    )
    def k(tab_hbm, t_hbm, c_hbm, src_hbm, dst_hbm, acc_hbm,
          cloc, srcv, dstv, tv, rows, acc_sh, sem):
        cid = lax.axis_index("c")
        sid = lax.axis_index("s")
        zv = jnp.zeros((16,), _f32)

        @pl.loop(0, K)
        def _(i):
            for j in range(HE // 16):
                rows[i, pl.ds(j * 16, 16)] = zv

        @pl.loop(0, ZR // K)
        def _(ci):
            pltpu.sync_copy(rows, acc_sh.at[pl.ds(sid * ZR + ci * K, K)])

        pltpu.sync_copy(c_hbm, cloc)
        plsc.subcore_barrier()

        base = (cid * NS + sid) * EPW
        c128 = jnp.full((16,), H, jnp.int32)

        @pl.loop(0, CH)
        def _(c):
            off = base + c * K
            pltpu.sync_copy(src_hbm.at[pl.ds(off, K)], srcv)
            pltpu.sync_copy(dst_hbm.at[pl.ds(off, K)], dstv)
            pltpu.sync_copy(t_hbm.at[pl.ds(off, K)], tv)
            pltpu.async_copy(tab_hbm.at[srcv], rows, sem).wait()

            @pl.loop(0, K // 16)
            def _(g):
                sl = pl.ds(g * 16, 16)
                ridx = lax.iota(jnp.int32, 16) + g * 16
                bv = plsc.load_gather(rows, [ridx, c128])
                cv = plsc.load_gather(cloc, [dstv[sl]])
                gs = tv[sl] + bv + cv
                w = jnp.exp(jnp.maximum(gs, 0.01 * gs))
                for i in range(16):
                    ws = w[i]
                    for j in range(HE // 16):
                        slj = pl.ds(j * 16, 16)
                        rows[g * 16 + i, slj] = rows[g * 16 + i, slj] * ws

            pltpu.sync_copy(rows, acc_sh.at[dstv], add=True)

        plsc.subcore_barrier()
        pltpu.sync_copy(acc_sh.at[pl.ds(sid * ZR, ZR)],
                        acc_hbm.at[cid, pl.ds(sid * ZR, ZR)])

    return k(table_ext, tscal, cscal, src, dst)


def kernel(x, edge_index, edge_attr, batch, W_embed, b_embed, W_lin1, b_lin1,
           gc_lin1, gc_lin2, gc_att_l, gc_att_r, gc_bias,
           gru1_Wih, gru1_Whh, gru1_bih, gru1_bhh,
           ac_lin, ac_att_src, ac_att_dst, ac_bias,
           gru2_Wih, gru2_Whh, gru2_bih, gru2_bhh,
           mc_lin, mc_att_src, mc_att_dst, mc_bias,
           mgru_Wih, mgru_Whh, mgru_bih, mgru_bhh,
           W_lin2, b_lin2, W_t1, b_t1, W_t2, b_t2):
    src = edge_index[0]
    dst = edge_index[1]
    xp = jnp.pad(x, ((0, NP - N), (0, 0)))
    batp = jnp.pad(batch, (0, NP - N), constant_values=G).reshape(NP, 1)

    row = lambda b: b.reshape(1, -1)

    xh, p, s_ext, r1 = _tc1(xp, W_embed.T, row(b_embed), W_lin1.T,
                            row(b_lin1), gc_lin1[:, :H].T, gc_lin2.T,
                            row(gc_att_r))

    pj = _sc_gather(p, src)

    t1 = _tc2(pj, edge_attr, gc_lin1[:, H:].T, row(gc_att_l))

    acc1 = _sc_aggregate(s_ext, t1.reshape(E), r1.reshape(NP), src, dst)

    xh2, xl_ext, v = _tc3(acc1, xh, row(gc_bias),
                          gru1_Wih.T, gru1_Whh.T, row(gru1_bih),
                          row(gru1_bhh),
                          ac_lin.T, row(ac_att_src), row(ac_att_dst))

    zeros_e = jnp.zeros((E,), _f32)
    acc2 = _sc_aggregate(xl_ext, zeros_e, v.reshape(NP), src, dst)

    xs, su, out0 = _tc4a(acc2, xh2, row(ac_bias),
                         gru2_Wih.T, gru2_Whh.T, row(gru2_bih), row(gru2_bhh),
                         mc_lin.T, row(mc_att_src), batp)

    out = out0
    for _ in range(2):
        out = _tc4b(out, xs, su, batp, mc_lin.T, row(mc_att_dst),
                    row(mc_bias), mgru_Wih.T, mgru_Whh.T, row(mgru_bih),
                    row(mgru_bhh))

    wt2T = jnp.zeros((64, H), _f32).at[:, 0].set(W_t2[0])
    bt2 = jnp.zeros((1, H), _f32).at[0, 0].set(b_t2[0])
    yfull = _tc4d(out, W_lin2.T, row(b_lin2), W_t1.T, row(b_t1), wt2T, bt2)
    return yfull[:, 0:1]


# speedup vs baseline: 11.9420x; 1.0740x over previous
import functools

import jax
import jax.numpy as jnp
from jax import lax
from jax.experimental import pallas as pl
from jax.experimental.pallas import tpu as pltpu
from jax.experimental.pallas import tpu_sc as plsc

N = 10000
NP = 10240
E = 320000
G = 256
H = 128
HE = 144
ED = 16

BLK = 1024
EB = 2560

NC = 2
NS = 16
NW = NC * NS
EPW = E // NW
K = 80
CH = EPW // K
ZR = NP // NS

_f32 = jnp.float32


def _leaky(x):
    return jnp.maximum(x, 0.01 * x)


def _elu(x):
    return jnp.where(x > 0, x, jnp.exp(jnp.minimum(x, 0.0)) - 1.0)


def _gru_math(xin, hprev, WihT, WhhT, bih, bhh):
    gi = jnp.dot(xin, WihT, preferred_element_type=_f32) + bih
    gh = jnp.dot(hprev, WhhT, preferred_element_type=_f32) + bhh
    r = jax.nn.sigmoid(gi[:, :H] + gh[:, :H])
    z = jax.nn.sigmoid(gi[:, H:2 * H] + gh[:, H:2 * H])
    n = jnp.tanh(gi[:, 2 * H:] + r * gh[:, 2 * H:])
    return (1.0 - z) * n + z * hprev


def _ext16(u):
    li = lax.broadcasted_iota(jnp.int32, (BLK, 16), 1)
    return jnp.where(li == 0, u, 0.0) + jnp.where(li == 1, 1.0, 0.0)


def _tc1_body(x_ref, weT, be, wlT, bl, a1T, g2T, ar,
              xh_ref, p_ref, s_ref, r_ref):
    xh0 = jnp.dot(x_ref[...], weT[...], preferred_element_type=_f32) + be[...]
    xh = _leaky(jnp.dot(xh0, wlT[...], preferred_element_type=_f32) + bl[...])
    xh_ref[...] = xh
    p_ref[...] = jnp.dot(xh, a1T[...], preferred_element_type=_f32)
    s_ref[:, :H] = jnp.dot(xh, g2T[...], preferred_element_type=_f32)
    s_ref[:, H:] = _ext16(jnp.zeros((BLK, 1), _f32))
    r_ref[...] = jnp.sum(xh * ar[...], axis=1, keepdims=True)


def _tc1(xp, weT, be, wlT, bl, a1T, g2T, ar):
    full = lambda s: pl.BlockSpec(s, lambda i: (0, 0))
    nblk = lambda s: pl.BlockSpec(s, lambda i: (i, 0))
    return pl.pallas_call(
        _tc1_body,
        grid=(NP // BLK,),
        in_specs=[nblk((BLK, H)), full((H, H)), full((1, H)), full((H, H)),
                  full((1, H)), full((H, H)), full((H, H)), full((1, H))],
        out_specs=[nblk((BLK, H)), nblk((BLK, H)), nblk((BLK, HE)),
                   nblk((BLK, 1))],
        out_shape=[jax.ShapeDtypeStruct((NP, H), _f32),
                   jax.ShapeDtypeStruct((NP, H), _f32),
                   jax.ShapeDtypeStruct((NP, HE), _f32),
                   jax.ShapeDtypeStruct((NP, 1), _f32)],
    )(xp, weT, be, wlT, bl, a1T, g2T, ar)


def _tc2_body(pj_ref, ea_ref, s_ref, d_ref, b2T, al, meta_ref):
    z = pj_ref[...] + jnp.dot(ea_ref[...], b2T[...],
                              preferred_element_type=_f32)
    m = _leaky(z)
    t = jnp.sum(m * al[...], axis=1, keepdims=True)
    meta_ref[:, 0:1] = lax.bitcast_convert_type(s_ref[...], _f32)
    meta_ref[:, 1:2] = lax.bitcast_convert_type(d_ref[...], _f32)
    meta_ref[:, 2:3] = t
    meta_ref[:, 3:4] = jnp.zeros((EB, 1), _f32)


def _tc2(pj, ea, srcc, dstc, b2T, al):
    full = lambda s: pl.BlockSpec(s, lambda i: (0, 0))
    eblk = lambda s: pl.BlockSpec(s, lambda i: (i, 0))
    return pl.pallas_call(
        _tc2_body,
        grid=(E // EB,),
        in_specs=[eblk((EB, H)), eblk((EB, ED)), eblk((EB, 1)),
                  eblk((EB, 1)), full((ED, H)), full((1, H))],
        out_specs=[eblk((EB, 4))],
        out_shape=[jax.ShapeDtypeStruct((E, 4), _f32)],
    )(pj, ea, srcc, dstc, b2T, al)[0]


def _tc3_body(acc_ref, xh_ref, gcb, wihT, whhT, bih, bhh,
              aclT, asrc, adst,
              xh2_ref, xl_ref, v_ref):
    hsum = acc_ref[0, :, :H] + acc_ref[1, :, :H]
    den = acc_ref[0, :, H + 1:H + 2] + acc_ref[1, :, H + 1:H + 2]
    h = _elu(hsum / (den + 1e-16) + gcb[...])
    xh2 = jnp.maximum(
        _gru_math(h, xh_ref[...], wihT[...], whhT[...], bih[...], bhh[...]),
        0.0)
    xh2_ref[...] = xh2
    xl = jnp.dot(xh2, aclT[...], preferred_element_type=_f32)
    xl_ref[:, :H] = xl
    xl_ref[:, H:] = _ext16(jnp.sum(xl * asrc[...], axis=1, keepdims=True))
    v_ref[...] = jnp.sum(xl * adst[...], axis=1, keepdims=True)


def _tc3(acc, xh, gcb, wihT, whhT, bih, bhh, aclT, asrc, adst):
    full = lambda s: pl.BlockSpec(s, lambda i: (0, 0))
    nblk = lambda s: pl.BlockSpec(s, lambda i: (i, 0))
    return pl.pallas_call(
        _tc3_body,
        grid=(NP // BLK,),
        in_specs=[pl.BlockSpec((NC, BLK, HE), lambda i: (0, i, 0)),
                  nblk((BLK, H)), full((1, H)),
                  full((H, 3 * H)), full((H, 3 * H)),
                  full((1, 3 * H)), full((1, 3 * H)),
                  full((H, H)), full((1, H)), full((1, H))],
        out_specs=[nblk((BLK, H)), nblk((BLK, HE)), nblk((BLK, 1))],
        out_shape=[jax.ShapeDtypeStruct((NP, H), _f32),
                   jax.ShapeDtypeStruct((NP, HE), _f32),
                   jax.ShapeDtypeStruct((NP, 1), _f32)],
    )(acc, xh, gcb, wihT, whhT, bih, bhh, aclT, asrc, adst)


def _tc4a_body(acc_ref, xh2_ref, acb, wihT, whhT, bih, bhh,
               mclT, msrc, bat_ref,
               xs_ref, su_ref, seg_ref):
    i = pl.program_id(0)
    ng = pl.num_programs(0)
    hsum = acc_ref[0, :, :H] + acc_ref[1, :, :H]
    den = acc_ref[0, :, H + 1:H + 2] + acc_ref[1, :, H + 1:H + 2]
    h2 = _elu(hsum / (den + 1e-16) + acb[...])
    xh3 = jnp.maximum(
        _gru_math(h2, xh2_ref[...], wihT[...], whhT[...], bih[...], bhh[...]),
        0.0)
    xs = jnp.dot(xh3, mclT[...], preferred_element_type=_f32)
    xs_ref[...] = xs
    su_ref[...] = jnp.sum(xs * msrc[...], axis=1, keepdims=True)
    gidx = lax.broadcasted_iota(jnp.int32, (BLK, G), 1)
    mask = (bat_ref[...] == gidx).astype(_f32)
    contrib = lax.dot_general(mask, xh3, (((0,), (0,)), ((), ())),
                              preferred_element_type=_f32)

    @pl.when(i == 0)
    def _():
        seg_ref[...] = contrib

    @pl.when(i > 0)
    def _():
        seg_ref[...] = seg_ref[...] + contrib

    @pl.when(i == ng - 1)
    def _():
        seg_ref[...] = jnp.maximum(seg_ref[...], 0.0)


def _tc4a(acc, xh2, acb, wihT, whhT, bih, bhh, mclT, msrc, batp):
    full = lambda s: pl.BlockSpec(s, lambda i: (0, 0))
    nblk = lambda s: pl.BlockSpec(s, lambda i: (i, 0))
    return pl.pallas_call(
        _tc4a_body,
        grid=(NP // BLK,),
        in_specs=[pl.BlockSpec((NC, BLK, HE), lambda i: (0, i, 0)),
                  nblk((BLK, H)), full((1, H)),
                  full((H, 3 * H)), full((H, 3 * H)),
                  full((1, 3 * H)), full((1, 3 * H)),
                  full((H, H)), full((1, H)), nblk((BLK, 1))],
        out_specs=[nblk((BLK, H)), nblk((BLK, 1)), full((G, H))],
        out_shape=[jax.ShapeDtypeStruct((NP, H), _f32),
                   jax.ShapeDtypeStruct((NP, 1), _f32),
                   jax.ShapeDtypeStruct((G, H), _f32)],
    )(acc, xh2, acb, wihT, whhT, bih, bhh, mclT, msrc, batp)


def _tc4b_body(out_ref, xs_ref, su_ref, bat_ref, mclT, mdst, mcb,
               wihT, whhT, bih, bhh,
               outnew_ref, num_s, den_s):
    i = pl.program_id(0)
    ng = pl.num_programs(0)
    outv = out_ref[...]
    od = jnp.dot(outv, mclT[...], preferred_element_type=_f32)
    sv = jnp.sum(od * mdst[...], axis=1, keepdims=True)
    gidx = lax.broadcasted_iota(jnp.int32, (BLK, G), 1)
    mask = (bat_ref[...] == gidx).astype(_f32)
    svn = lax.dot_general(mask, sv, (((1,), (0,)), ((), ())),
                          preferred_element_type=_f32)
    w = jnp.exp(_leaky(su_ref[...] + svn))
    nc = lax.dot_general(mask, w * xs_ref[...], (((0,), (0,)), ((), ())),
                         preferred_element_type=_f32)
    dc = lax.dot_general(mask, jnp.broadcast_to(w, (BLK, H)),
                         (((0,), (0,)), ((), ())),
                         preferred_element_type=_f32)

    @pl.when(i == 0)
    def _():
        num_s[...] = nc
        den_s[...] = dc

    @pl.when(i > 0)
    def _():
        num_s[...] = num_s[...] + nc
        den_s[...] = den_s[...] + dc

    @pl.when(i == ng - 1)
    def _():
        hm = _elu(num_s[...] / (den_s[...] + 1e-16) + mcb[...])
        outnew_ref[...] = jnp.maximum(
            _gru_math(hm, outv, wihT[...], whhT[...], bih[...], bhh[...]),
            0.0)


def _tc4b(out, xs, su, batp, mclT, mdst, mcb, wihT, whhT, bih, bhh):
    full = lambda s: pl.BlockSpec(s, lambda i: (0, 0))
    nblk = lambda s: pl.BlockSpec(s, lambda i: (i, 0))
    return pl.pallas_call(
        _tc4b_body,
        grid=(NP // BLK,),
        in_specs=[full((G, H)), nblk((BLK, H)), nblk((BLK, 1)),
                  nblk((BLK, 1)), full((H, H)), full((1, H)), full((1, H)),
                  full((H, 3 * H)), full((H, 3 * H)),
                  full((1, 3 * H)), full((1, 3 * H))],
        out_specs=[full((G, H))],
        out_shape=[jax.ShapeDtypeStruct((G, H), _f32)],
        scratch_shapes=[pltpu.VMEM((G, H), _f32), pltpu.VMEM((G, H), _f32)],
    )(out, xs, su, batp, mclT, mdst, mcb, wihT, whhT, bih, bhh)[0]


def _tc4d_body(out_ref, wl2T, bl2, wt1T, bt1, wt2T, bt2, y_ref):
    fp = jnp.dot(out_ref[...], wl2T[...], preferred_element_type=_f32) \
        + bl2[...]
    hh = jnp.maximum(
        jnp.dot(fp, wt1T[...], preferred_element_type=_f32) + bt1[...], 0.0)
    y_ref[...] = jnp.dot(hh, wt2T[...], preferred_element_type=_f32) \
        + bt2[...]


def _tc4d(out, wl2T, bl2, wt1T, bt1, wt2T, bt2):
    full = lambda s: pl.BlockSpec(s, lambda: (0, 0))
    return pl.pallas_call(
        _tc4d_body,
        in_specs=[full((G, H)), full((H, H)), full((1, H)),
                  full((H, 64)), full((1, 64)), full((64, H)), full((1, H))],
        out_specs=full((G, H)),
        out_shape=jax.ShapeDtypeStruct((G, H), _f32),
    )(out, wl2T, bl2, wt1T, bt1, wt2T, bt2)


def _sc_mesh():
    return plsc.VectorSubcoreMesh(core_axis_name="c", subcore_axis_name="s")


_SC_PARAMS = pltpu.CompilerParams(needs_layout_passes=False,
                                  use_tc_tiling_on_sc=False)


def _sc_gather(table, idx):
    @functools.partial(
        pl.kernel,
        out_type=jax.ShapeDtypeStruct((E, H), _f32),
        mesh=_sc_mesh(),
        compiler_params=_SC_PARAMS,
        scratch_types=[pltpu.VMEM((K,), jnp.int32),
                       pltpu.VMEM((K,), jnp.int32),
                       pltpu.VMEM((K, H), _f32),
                       pltpu.VMEM((K, H), _f32),
                       pltpu.SemaphoreType.DMA,
                       pltpu.SemaphoreType.DMA],
    )
    def k(tab_hbm, idx_hbm, out_hbm, idx0, idx1, rows0, rows1, g0, g1):
        cid = lax.axis_index("c")
        sid = lax.axis_index("s")
        base = (cid * NS + sid) * EPW

        pltpu.sync_copy(idx_hbm.at[pl.ds(base, K)], idx0)
        pltpu.async_copy(tab_hbm.at[idx0], rows0, g0)

        @pl.loop(0, CH - 1, step=2)
        def _(c):
            off = base + c * K
            pltpu.sync_copy(idx_hbm.at[pl.ds(off + K, K)], idx1)
            pltpu.async_copy(tab_hbm.at[idx1], rows1, g1)
            pltpu.make_async_copy(tab_hbm.at[idx0], rows0, g0).wait()
            pltpu.sync_copy(rows0, out_hbm.at[pl.ds(off, K)])

            @pl.when(c + 2 < CH)
            def _():
                pltpu.sync_copy(idx_hbm.at[pl.ds(off + 2 * K, K)], idx0)
                pltpu.async_copy(tab_hbm.at[idx0], rows0, g0)

            pltpu.make_async_copy(tab_hbm.at[idx1], rows1, g1).wait()
            pltpu.sync_copy(rows1, out_hbm.at[pl.ds(off + K, K)])

        pltpu.make_async_copy(tab_hbm.at[idx0], rows0, g0).wait()
        pltpu.sync_copy(rows0, out_hbm.at[pl.ds(base + (CH - 1) * K, K)])

    return k(table, idx)


K4 = K * 4


def _sc_aggregate(table_ext, meta, cscal, tcol):
    @functools.partial(
        pl.kernel,
        out_type=jax.ShapeDtypeStruct((NC, NP, HE), _f32),
        mesh=_sc_mesh(),
        compiler_params=_SC_PARAMS,
        scratch_types=[pltpu.VMEM((NP,), _f32),
                       pltpu.VMEM((K4,), _f32),
                       pltpu.VMEM((K4,), _f32),
                       pltpu.VMEM((K,), jnp.int32),
                       pltpu.VMEM((K,), jnp.int32),
                       pltpu.VMEM((K,), _f32),
                       pltpu.VMEM((K, HE), _f32),
                       pltpu.VMEM_SHARED((NP, HE), _f32),
                       pltpu.SemaphoreType.DMA,
                       pltpu.SemaphoreType.DMA,
                       pltpu.SemaphoreType.DMA],
    )
    def k(tab_hbm, meta_hbm, c_hbm, acc_hbm,
          cloc, meta0, meta1, srcv, dstv, tv, rows, acc_sh, sm0, sm1, sg):
        cid = lax.axis_index("c")
        sid = lax.axis_index("s")
        zv = jnp.zeros((16,), _f32)

        @pl.loop(0, K)
        def _(i):
            for j in range(HE // 16):
                rows[i, pl.ds(j * 16, 16)] = zv

        @pl.loop(0, ZR // K)
        def _(ci):
            pltpu.sync_copy(rows, acc_sh.at[pl.ds(sid * ZR + ci * K, K)])

        pltpu.sync_copy(c_hbm, cloc)
        plsc.subcore_barrier()

        base4 = (cid * NS + sid) * EPW * 4
        c128 = jnp.full((16,), H, jnp.int32)

        def chunk(c, mcur, scur, mnext, snext, prefetch):
            pltpu.make_async_copy(meta_hbm.at[pl.ds(base4 + c * K4, K4)],
                                  mcur, scur).wait()
            if prefetch:
                pltpu.async_copy(
                    meta_hbm.at[pl.ds(base4 + (c + 1) * K4, K4)],
                    mnext, snext)

            @pl.loop(0, K // 16)
            def _(g):
                sl = pl.ds(g * 16, 16)
                i4 = (lax.iota(jnp.int32, 16) + g * 16) * 4
                srcv[sl] = plsc.bitcast(plsc.load_gather(mcur, [i4]),
                                        jnp.int32)
                dstv[sl] = plsc.bitcast(plsc.load_gather(mcur, [i4 + 1]),
                                        jnp.int32)
                tv[sl] = plsc.load_gather(mcur, [i4 + tcol])

            pltpu.async_copy(tab_hbm.at[srcv], rows, sg).wait()

            @pl.loop(0, K // 16)
            def _(g):
                sl = pl.ds(g * 16, 16)
                ridx = lax.iota(jnp.int32, 16) + g * 16
                bv = plsc.load_gather(rows, [ridx, c128])
                cv = plsc.load_gather(cloc, [dstv[sl]])
                gs = tv[sl] + bv + cv
                w = jnp.exp(jnp.maximum(gs, 0.01 * gs))
                for i in range(16):
                    ws = w[i]
                    for j in range(HE // 16):
                        slj = pl.ds(j * 16, 16)
                        rows[g * 16 + i, slj] = rows[g * 16 + i, slj] * ws

            pltpu.sync_copy(rows, acc_sh.at[dstv], add=True)

        pltpu.async_copy(meta_hbm.at[pl.ds(base4, K4)], meta0, sm0)

        @pl.loop(0, CH - 1, step=2)
        def _(c):
            chunk(c, meta0, sm0, meta1, sm1, True)
            chunk(c + 1, meta1, sm1, meta0, sm0, True)

        chunk(CH - 1, meta0, sm0, meta1, sm1, False)

        plsc.subcore_barrier()
        pltpu.sync_copy(acc_sh.at[pl.ds(sid * ZR, ZR)],
                        acc_hbm.at[cid, pl.ds(sid * ZR, ZR)])

    return k(table_ext, meta, cscal)


def kernel(x, edge_index, edge_attr, batch, W_embed, b_embed, W_lin1, b_lin1,
           gc_lin1, gc_lin2, gc_att_l, gc_att_r, gc_bias,
           gru1_Wih, gru1_Whh, gru1_bih, gru1_bhh,
           ac_lin, ac_att_src, ac_att_dst, ac_bias,
           gru2_Wih, gru2_Whh, gru2_bih, gru2_bhh,
           mc_lin, mc_att_src, mc_att_dst, mc_bias,
           mgru_Wih, mgru_Whh, mgru_bih, mgru_bhh,
           W_lin2, b_lin2, W_t1, b_t1, W_t2, b_t2):
    src = edge_index[0]
    dst = edge_index[1]
    xp = jnp.pad(x, ((0, NP - N), (0, 0)))
    batp = jnp.pad(batch, (0, NP - N), constant_values=G).reshape(NP, 1)

    row = lambda b: b.reshape(1, -1)

    xh, p, s_ext, r1 = _tc1(xp, W_embed.T, row(b_embed), W_lin1.T,
                            row(b_lin1), gc_lin1[:, :H].T, gc_lin2.T,
                            row(gc_att_r))

    pj = _sc_gather(p, src)

    meta = _tc2(pj, edge_attr, src.reshape(E, 1), dst.reshape(E, 1),
                gc_lin1[:, H:].T, row(gc_att_l)).reshape(E * 4)

    acc1 = _sc_aggregate(s_ext, meta, r1.reshape(NP), 2)

    xh2, xl_ext, v = _tc3(acc1, xh, row(gc_bias),
                          gru1_Wih.T, gru1_Whh.T, row(gru1_bih),
                          row(gru1_bhh),
                          ac_lin.T, row(ac_att_src), row(ac_att_dst))

    acc2 = _sc_aggregate(xl_ext, meta, v.reshape(NP), 3)

    xs, su, out0 = _tc4a(acc2, xh2, row(ac_bias),
                         gru2_Wih.T, gru2_Whh.T, row(gru2_bih), row(gru2_bhh),
                         mc_lin.T, row(mc_att_src), batp)

    out = out0
    for _ in range(2):
        out = _tc4b(out, xs, su, batp, mc_lin.T, row(mc_att_dst),
                    row(mc_bias), mgru_Wih.T, mgru_Whh.T, row(mgru_bih),
                    row(mgru_bhh))

    wt2T = jnp.zeros((64, H), _f32).at[:, 0].set(W_t2[0])
    bt2 = jnp.zeros((1, H), _f32).at[0, 0].set(b_t2[0])
    yfull = _tc4d(out, W_lin2.T, row(b_lin2), W_t1.T, row(b_t1), wt2T, bt2)
    return yfull[:, 0:1]


# speedup vs baseline: 14.4701x; 1.2117x over previous
import functools

import jax
import jax.numpy as jnp
from jax import lax
from jax.experimental import pallas as pl
from jax.experimental.pallas import tpu as pltpu
from jax.experimental.pallas import tpu_sc as plsc

N = 10000
NP = 10240
E = 320000
G = 256
H = 128
HE = 144
ED = 16

BLK = 1024
EP = 327680
EB = 2048

NC = 2
NS = 16
NW = NC * NS
EPW = E // NW
K = 80
CH = EPW // K
ZR = NP // NS

_f32 = jnp.float32


def _leaky(x):
    return jnp.maximum(x, 0.01 * x)


def _elu(x):
    return jnp.where(x > 0, x, jnp.exp(jnp.minimum(x, 0.0)) - 1.0)


def _gru_math(xin, hprev, WihT, WhhT, bih, bhh):
    gi = jnp.dot(xin, WihT, preferred_element_type=_f32) + bih
    gh = jnp.dot(hprev, WhhT, preferred_element_type=_f32) + bhh
    r = jax.nn.sigmoid(gi[:, :H] + gh[:, :H])
    z = jax.nn.sigmoid(gi[:, H:2 * H] + gh[:, H:2 * H])
    n = jnp.tanh(gi[:, 2 * H:] + r * gh[:, 2 * H:])
    return (1.0 - z) * n + z * hprev


def _ext16(u):
    li = lax.broadcasted_iota(jnp.int32, (BLK, 16), 1)
    return jnp.where(li == 0, u, 0.0) + jnp.where(li == 1, 1.0, 0.0)


def _tc1_body(x_ref, weT, be, wlT, bl, a1T, g2T, ar,
              xh_ref, p_ref, s_ref, r_ref):
    xh0 = jnp.dot(x_ref[...], weT[...], preferred_element_type=_f32) + be[...]
    xh = _leaky(jnp.dot(xh0, wlT[...], preferred_element_type=_f32) + bl[...])
    xh_ref[...] = xh
    p_ref[...] = jnp.dot(xh, a1T[...], preferred_element_type=_f32)
    s_ref[:, :H] = jnp.dot(xh, g2T[...], preferred_element_type=_f32)
    s_ref[:, H:] = _ext16(jnp.zeros((BLK, 1), _f32))
    r_ref[...] = jnp.sum(xh * ar[...], axis=1)


def _tc1(xp, weT, be, wlT, bl, a1T, g2T, ar):
    full = lambda s: pl.BlockSpec(s, lambda i: (0, 0))
    nblk = lambda s: pl.BlockSpec(s, lambda i: (i, 0))
    return pl.pallas_call(
        _tc1_body,
        grid=(NP // BLK,),
        in_specs=[nblk((BLK, H)), full((H, H)), full((1, H)), full((H, H)),
                  full((1, H)), full((H, H)), full((H, H)), full((1, H))],
        out_specs=[nblk((BLK, H)), nblk((BLK, H)), nblk((BLK, HE)),
                   pl.BlockSpec((BLK,), lambda i: (i,))],
        out_shape=[jax.ShapeDtypeStruct((NP, H), _f32),
                   jax.ShapeDtypeStruct((NP, H), _f32),
                   jax.ShapeDtypeStruct((NP, HE), _f32),
                   jax.ShapeDtypeStruct((NP,), _f32)],
    )(xp, weT, be, wlT, bl, a1T, g2T, ar)


def _tc2_body(pj_ref, ea_ref, b2T, al, t_ref):
    z = pj_ref[...] + jnp.dot(ea_ref[...], b2T[...],
                              preferred_element_type=_f32)
    m = _leaky(z)
    t_ref[...] = jnp.sum(m * al[...], axis=1)


def _tc2(pj, ea, b2T, al):
    full = lambda s: pl.BlockSpec(s, lambda i: (0, 0))
    eblk = lambda s: pl.BlockSpec(s, lambda i: (i, 0))
    return pl.pallas_call(
        _tc2_body,
        grid=(EP // EB,),
        in_specs=[eblk((EB, H)), eblk((EB, ED)), full((ED, H)), full((1, H))],
        out_specs=[pl.BlockSpec((EB,), lambda i: (i,))],
        out_shape=[jax.ShapeDtypeStruct((EP,), _f32)],
    )(pj, ea, b2T, al)[0]


def _tc3_body(acc_ref, xh_ref, gcb, wihT, whhT, bih, bhh,
              aclT, asrc, adst,
              xh2_ref, xl_ref, v_ref):
    hsum = acc_ref[0, :, :H] + acc_ref[1, :, :H]
    den = acc_ref[0, :, H + 1:H + 2] + acc_ref[1, :, H + 1:H + 2]
    h = _elu(hsum / (den + 1e-16) + gcb[...])
    xh2 = jnp.maximum(
        _gru_math(h, xh_ref[...], wihT[...], whhT[...], bih[...], bhh[...]),
        0.0)
    xh2_ref[...] = xh2
    xl = jnp.dot(xh2, aclT[...], preferred_element_type=_f32)
    xl_ref[:, :H] = xl
    xl_ref[:, H:] = _ext16(jnp.sum(xl * asrc[...], axis=1, keepdims=True))
    v_ref[...] = jnp.sum(xl * adst[...], axis=1)


def _tc3(acc, xh, gcb, wihT, whhT, bih, bhh, aclT, asrc, adst):
    full = lambda s: pl.BlockSpec(s, lambda i: (0, 0))
    nblk = lambda s: pl.BlockSpec(s, lambda i: (i, 0))
    return pl.pallas_call(
        _tc3_body,
        grid=(NP // BLK,),
        in_specs=[pl.BlockSpec((NC, BLK, HE), lambda i: (0, i, 0)),
                  nblk((BLK, H)), full((1, H)),
                  full((H, 3 * H)), full((H, 3 * H)),
                  full((1, 3 * H)), full((1, 3 * H)),
                  full((H, H)), full((1, H)), full((1, H))],
        out_specs=[nblk((BLK, H)), nblk((BLK, HE)),
                   pl.BlockSpec((BLK,), lambda i: (i,))],
        out_shape=[jax.ShapeDtypeStruct((NP, H), _f32),
                   jax.ShapeDtypeStruct((NP, HE), _f32),
                   jax.ShapeDtypeStruct((NP,), _f32)],
    )(acc, xh, gcb, wihT, whhT, bih, bhh, aclT, asrc, adst)


def _tc4a_body(acc_ref, xh2_ref, acb, wihT, whhT, bih, bhh,
               mclT, msrc, bat_ref,
               xs_ref, su_ref, seg_ref):
    i = pl.program_id(0)
    ng = pl.num_programs(0)
    hsum = acc_ref[0, :, :H] + acc_ref[1, :, :H]
    den = acc_ref[0, :, H + 1:H + 2] + acc_ref[1, :, H + 1:H + 2]
    h2 = _elu(hsum / (den + 1e-16) + acb[...])
    xh3 = jnp.maximum(
        _gru_math(h2, xh2_ref[...], wihT[...], whhT[...], bih[...], bhh[...]),
        0.0)
    xs = jnp.dot(xh3, mclT[...], preferred_element_type=_f32)
    xs_ref[...] = xs
    su_ref[...] = jnp.sum(xs * msrc[...], axis=1, keepdims=True)
    gidx = lax.broadcasted_iota(jnp.int32, (BLK, G), 1)
    mask = (bat_ref[...] == gidx).astype(_f32)
    contrib = lax.dot_general(mask, xh3, (((0,), (0,)), ((), ())),
                              preferred_element_type=_f32)

    @pl.when(i == 0)
    def _():
        seg_ref[...] = contrib

    @pl.when(i > 0)
    def _():
        seg_ref[...] = seg_ref[...] + contrib

    @pl.when(i == ng - 1)
    def _():
        seg_ref[...] = jnp.maximum(seg_ref[...], 0.0)


def _tc4a(acc, xh2, acb, wihT, whhT, bih, bhh, mclT, msrc, batp):
    full = lambda s: pl.BlockSpec(s, lambda i: (0, 0))
    nblk = lambda s: pl.BlockSpec(s, lambda i: (i, 0))
    return pl.pallas_call(
        _tc4a_body,
        grid=(NP // BLK,),
        in_specs=[pl.BlockSpec((NC, BLK, HE), lambda i: (0, i, 0)),
                  nblk((BLK, H)), full((1, H)),
                  full((H, 3 * H)), full((H, 3 * H)),
                  full((1, 3 * H)), full((1, 3 * H)),
                  full((H, H)), full((1, H)), nblk((BLK, 1))],
        out_specs=[nblk((BLK, H)), nblk((BLK, 1)), full((G, H))],
        out_shape=[jax.ShapeDtypeStruct((NP, H), _f32),
                   jax.ShapeDtypeStruct((NP, 1), _f32),
                   jax.ShapeDtypeStruct((G, H), _f32)],
    )(acc, xh2, acb, wihT, whhT, bih, bhh, mclT, msrc, batp)


def _tc4b_body(out_ref, xs_ref, su_ref, bat_ref, mclT, mdst, mcb,
               wihT, whhT, bih, bhh,
               outnew_ref, num_s, den_s):
    i = pl.program_id(0)
    ng = pl.num_programs(0)
    outv = out_ref[...]
    od = jnp.dot(outv, mclT[...], preferred_element_type=_f32)
    sv = jnp.sum(od * mdst[...], axis=1, keepdims=True)
    gidx = lax.broadcasted_iota(jnp.int32, (BLK, G), 1)
    mask = (bat_ref[...] == gidx).astype(_f32)
    svn = lax.dot_general(mask, sv, (((1,), (0,)), ((), ())),
                          preferred_element_type=_f32)
    w = jnp.exp(_leaky(su_ref[...] + svn))
    nc = lax.dot_general(mask, w * xs_ref[...], (((0,), (0,)), ((), ())),
                         preferred_element_type=_f32)
    dc = lax.dot_general(mask, jnp.broadcast_to(w, (BLK, H)),
                         (((0,), (0,)), ((), ())),
                         preferred_element_type=_f32)

    @pl.when(i == 0)
    def _():
        num_s[...] = nc
        den_s[...] = dc

    @pl.when(i > 0)
    def _():
        num_s[...] = num_s[...] + nc
        den_s[...] = den_s[...] + dc

    @pl.when(i == ng - 1)
    def _():
        hm = _elu(num_s[...] / (den_s[...] + 1e-16) + mcb[...])
        outnew_ref[...] = jnp.maximum(
            _gru_math(hm, outv, wihT[...], whhT[...], bih[...], bhh[...]),
            0.0)


def _tc4b(out, xs, su, batp, mclT, mdst, mcb, wihT, whhT, bih, bhh):
    full = lambda s: pl.BlockSpec(s, lambda i: (0, 0))
    nblk = lambda s: pl.BlockSpec(s, lambda i: (i, 0))
    return pl.pallas_call(
        _tc4b_body,
        grid=(NP // BLK,),
        in_specs=[full((G, H)), nblk((BLK, H)), nblk((BLK, 1)),
                  nblk((BLK, 1)), full((H, H)), full((1, H)), full((1, H)),
                  full((H, 3 * H)), full((H, 3 * H)),
                  full((1, 3 * H)), full((1, 3 * H))],
        out_specs=[full((G, H))],
        out_shape=[jax.ShapeDtypeStruct((G, H), _f32)],
        scratch_shapes=[pltpu.VMEM((G, H), _f32), pltpu.VMEM((G, H), _f32)],
    )(out, xs, su, batp, mclT, mdst, mcb, wihT, whhT, bih, bhh)[0]


def _tc4d_body(out_ref, wl2T, bl2, wt1T, bt1, wt2T, bt2, y_ref):
    fp = jnp.dot(out_ref[...], wl2T[...], preferred_element_type=_f32) \
        + bl2[...]
    hh = jnp.maximum(
        jnp.dot(fp, wt1T[...], preferred_element_type=_f32) + bt1[...], 0.0)
    y_ref[...] = jnp.dot(hh, wt2T[...], preferred_element_type=_f32) \
        + bt2[...]


def _tc4d(out, wl2T, bl2, wt1T, bt1, wt2T, bt2):
    full = lambda s: pl.BlockSpec(s, lambda: (0, 0))
    return pl.pallas_call(
        _tc4d_body,
        in_specs=[full((G, H)), full((H, H)), full((1, H)),
                  full((H, 64)), full((1, 64)), full((64, H)), full((1, H))],
        out_specs=full((G, H)),
        out_shape=jax.ShapeDtypeStruct((G, H), _f32),
    )(out, wl2T, bl2, wt1T, bt1, wt2T, bt2)


def _sc_mesh():
    return plsc.VectorSubcoreMesh(core_axis_name="c", subcore_axis_name="s")


_SC_PARAMS = pltpu.CompilerParams(needs_layout_passes=False,
                                  use_tc_tiling_on_sc=False)
_SC_PARAMS_TILED = pltpu.CompilerParams()


def _sc_gather(table, idx):
    @functools.partial(
        pl.kernel,
        out_type=jax.ShapeDtypeStruct((EP, H), _f32),
        mesh=_sc_mesh(),
        compiler_params=_SC_PARAMS_TILED,
        scratch_types=[pltpu.VMEM((K,), jnp.int32),
                       pltpu.VMEM((K,), jnp.int32),
                       pltpu.VMEM((K, H), _f32),
                       pltpu.VMEM((K, H), _f32),
                       pltpu.SemaphoreType.DMA,
                       pltpu.SemaphoreType.DMA],
    )
    def k(tab_hbm, idx_hbm, out_hbm, idx0, idx1, rows0, rows1, g0, g1):
        cid = lax.axis_index("c")
        sid = lax.axis_index("s")
        base = (cid * NS + sid) * EPW

        pltpu.sync_copy(idx_hbm.at[pl.ds(base, K)], idx0)
        pltpu.async_copy(tab_hbm.at[idx0], rows0, g0)

        @pl.loop(0, CH - 1, step=2)
        def _(c):
            off = base + c * K
            pltpu.sync_copy(idx_hbm.at[pl.ds(off + K, K)], idx1)
            pltpu.async_copy(tab_hbm.at[idx1], rows1, g1)
            pltpu.make_async_copy(tab_hbm.at[idx0], rows0, g0).wait()
            pltpu.sync_copy(rows0, out_hbm.at[pl.ds(off, K)])

            @pl.when(c + 2 < CH)
            def _():
                pltpu.sync_copy(idx_hbm.at[pl.ds(off + 2 * K, K)], idx0)
                pltpu.async_copy(tab_hbm.at[idx0], rows0, g0)

            pltpu.make_async_copy(tab_hbm.at[idx1], rows1, g1).wait()
            pltpu.sync_copy(rows1, out_hbm.at[pl.ds(off + K, K)])

        pltpu.make_async_copy(tab_hbm.at[idx0], rows0, g0).wait()
        pltpu.sync_copy(rows0, out_hbm.at[pl.ds(base + (CH - 1) * K, K)])

    return k(table, idx)


def _sc_aggregate(table_ext, tscal, cscal, src, dst):
    @functools.partial(
        pl.kernel,
        out_type=jax.ShapeDtypeStruct((NC, NP, HE), _f32),
        mesh=_sc_mesh(),
        compiler_params=_SC_PARAMS,
        scratch_types=[pltpu.VMEM((NP,), _f32),
                       pltpu.VMEM((K,), jnp.int32),
                       pltpu.VMEM((K,), jnp.int32),
                       pltpu.VMEM((K,), jnp.int32),
                       pltpu.VMEM((K,), jnp.int32),
                       pltpu.VMEM((K,), _f32),
                       pltpu.VMEM((K,), _f32),
                       pltpu.VMEM((K, HE), _f32),
                       pltpu.VMEM_SHARED((NP, HE), _f32),
                       pltpu.SemaphoreType.DMA,
                       pltpu.SemaphoreType.DMA,
                       pltpu.SemaphoreType.DMA],
    )
    def k(tab_hbm, t_hbm, c_hbm, src_hbm, dst_hbm, acc_hbm,
          cloc, src0, src1, dst0, dst1, tv0, tv1, rows, acc_sh,
          sm0, sm1, sg):
        cid = lax.axis_index("c")
        sid = lax.axis_index("s")
        zv = jnp.zeros((16,), _f32)

        @pl.loop(0, K)
        def _(i):
            for j in range(HE // 16):
                rows[i, pl.ds(j * 16, 16)] = zv

        @pl.loop(0, ZR // K)
        def _(ci):
            pltpu.sync_copy(rows, acc_sh.at[pl.ds(sid * ZR + ci * K, K)])

        pltpu.sync_copy(c_hbm, cloc)
        plsc.subcore_barrier()

        base = (cid * NS + sid) * EPW
        c128 = jnp.full((16,), H, jnp.int32)

        def fetch(c, srcb, dstb, tb, sem):
            off = base + c * K
            pltpu.async_copy(src_hbm.at[pl.ds(off, K)], srcb, sem)
            pltpu.async_copy(dst_hbm.at[pl.ds(off, K)], dstb, sem)
            pltpu.async_copy(t_hbm.at[pl.ds(off, K)], tb, sem)

        def fetch_wait(c, srcb, dstb, tb, sem):
            off = base + c * K
            pltpu.make_async_copy(src_hbm.at[pl.ds(off, K)], srcb,
                                  sem).wait()
            pltpu.make_async_copy(dst_hbm.at[pl.ds(off, K)], dstb,
                                  sem).wait()
            pltpu.make_async_copy(t_hbm.at[pl.ds(off, K)], tb, sem).wait()

        def chunk(c, srcb, dstb, tb, scur, nsrc, ndst, nt, snext, prefetch):
            fetch_wait(c, srcb, dstb, tb, scur)
            if prefetch:
                fetch(c + 1, nsrc, ndst, nt, snext)

            pltpu.async_copy(tab_hbm.at[srcb], rows, sg).wait()

            @pl.loop(0, K // 16)
            def _(g):
                sl = pl.ds(g * 16, 16)
                ridx = lax.iota(jnp.int32, 16) + g * 16
                bv = plsc.load_gather(rows, [ridx, c128])
                cv = plsc.load_gather(cloc, [dstb[sl]])
                gs = tb[sl] + bv + cv
                w = jnp.exp(jnp.maximum(gs, 0.01 * gs))
                for i in range(16):
                    ws = w[i]
                    for j in range(HE // 16):
                        slj = pl.ds(j * 16, 16)
                        rows[g * 16 + i, slj] = rows[g * 16 + i, slj] * ws

            pltpu.sync_copy(rows, acc_sh.at[dstb], add=True)

        fetch(0, src0, dst0, tv0, sm0)

        @pl.loop(0, CH - 1, step=2)
        def _(c):
            chunk(c, src0, dst0, tv0, sm0, src1, dst1, tv1, sm1, True)
            chunk(c + 1, src1, dst1, tv1, sm1, src0, dst0, tv0, sm0, True)

        chunk(CH - 1, src0, dst0, tv0, sm0, src1, dst1, tv1, sm1, False)

        plsc.subcore_barrier()
        pltpu.sync_copy(acc_sh.at[pl.ds(sid * ZR, ZR)],
                        acc_hbm.at[cid, pl.ds(sid * ZR, ZR)])

    return k(table_ext, tscal, cscal, src, dst)


def kernel(x, edge_index, edge_attr, batch, W_embed, b_embed, W_lin1, b_lin1,
           gc_lin1, gc_lin2, gc_att_l, gc_att_r, gc_bias,
           gru1_Wih, gru1_Whh, gru1_bih, gru1_bhh,
           ac_lin, ac_att_src, ac_att_dst, ac_bias,
           gru2_Wih, gru2_Whh, gru2_bih, gru2_bhh,
           mc_lin, mc_att_src, mc_att_dst, mc_bias,
           mgru_Wih, mgru_Whh, mgru_bih, mgru_bhh,
           W_lin2, b_lin2, W_t1, b_t1, W_t2, b_t2):
    src = edge_index[0]
    dst = edge_index[1]
    xp = jnp.pad(x, ((0, NP - N), (0, 0)))
    batp = jnp.pad(batch, (0, NP - N), constant_values=G).reshape(NP, 1)

    row = lambda b: b.reshape(1, -1)

    xh, p, s_ext, r1 = _tc1(xp, W_embed.T, row(b_embed), W_lin1.T,
                            row(b_lin1), gc_lin1[:, :H].T, gc_lin2.T,
                            row(gc_att_r))

    pj = _sc_gather(p, src)

    eap = jnp.pad(edge_attr, ((0, EP - E), (0, 0)))
    t1 = _tc2(pj, eap, gc_lin1[:, H:].T, row(gc_att_l))[:E]

    acc1 = _sc_aggregate(s_ext, t1, r1, src, dst)

    xh2, xl_ext, v = _tc3(acc1, xh, row(gc_bias),
                          gru1_Wih.T, gru1_Whh.T, row(gru1_bih),
                          row(gru1_bhh),
                          ac_lin.T, row(ac_att_src), row(ac_att_dst))

    zeros_e = jnp.zeros((E,), _f32)
    acc2 = _sc_aggregate(xl_ext, zeros_e, v, src, dst)

    xs, su, out0 = _tc4a(acc2, xh2, row(ac_bias),
                         gru2_Wih.T, gru2_Whh.T, row(gru2_bih), row(gru2_bhh),
                         mc_lin.T, row(mc_att_src), batp)

    out = out0
    for _ in range(2):
        out = _tc4b(out, xs, su, batp, mc_lin.T, row(mc_att_dst),
                    row(mc_bias), mgru_Wih.T, mgru_Whh.T, row(mgru_bih),
                    row(mgru_bhh))

    wt2T = jnp.zeros((64, H), _f32).at[:, 0].set(W_t2[0])
    bt2 = jnp.zeros((1, H), _f32).at[0, 0].set(b_t2[0])
    yfull = _tc4d(out, W_lin2.T, row(b_lin2), W_t1.T, row(b_t1), wt2T, bt2)
    return yfull[:, 0:1]


# speedup vs baseline: 16.3945x; 1.1330x over previous
import functools

import jax
import jax.numpy as jnp
from jax import lax
from jax.experimental import pallas as pl
from jax.experimental.pallas import tpu as pltpu
from jax.experimental.pallas import tpu_sc as plsc

N = 10000
NP = 10240
E = 320000
G = 256
H = 128
HE = 144
ED = 16

BLK = 1024
EP = 327680
EB = 2048

NC = 2
NS = 16
NW = NC * NS
EPW = E // NW
K = 80
CH = EPW // K
ZR = NP // NS

_f32 = jnp.float32
_PREC = lax.Precision.DEFAULT


def _leaky(x):
    return jnp.maximum(x, 0.01 * x)


def _elu(x):
    return jnp.where(x > 0, x, jnp.exp(jnp.minimum(x, 0.0)) - 1.0)


def _gru_math(xin, hprev, WihT, WhhT, bih, bhh):
    gi = jnp.dot(xin, WihT, preferred_element_type=_f32, precision=_PREC) + bih
    gh = jnp.dot(hprev, WhhT, preferred_element_type=_f32, precision=_PREC) + bhh
    r = jax.nn.sigmoid(gi[:, :H] + gh[:, :H])
    z = jax.nn.sigmoid(gi[:, H:2 * H] + gh[:, H:2 * H])
    n = jnp.tanh(gi[:, 2 * H:] + r * gh[:, 2 * H:])
    return (1.0 - z) * n + z * hprev


def _ext16(u):
    li = lax.broadcasted_iota(jnp.int32, (BLK, 16), 1)
    return jnp.where(li == 0, u, 0.0) + jnp.where(li == 1, 1.0, 0.0)


def _tc1_body(x_ref, weT, be, wlT, bl, a1T, g2T, ar,
              xh_ref, p_ref, s_ref, r_ref):
    xh0 = jnp.dot(x_ref[...], weT[...], preferred_element_type=_f32, precision=_PREC) + be[...]
    xh = _leaky(jnp.dot(xh0, wlT[...], preferred_element_type=_f32, precision=_PREC) + bl[...])
    xh_ref[...] = xh
    p_ref[...] = jnp.dot(xh, a1T[...], preferred_element_type=_f32, precision=_PREC)
    s_ref[:, :H] = jnp.dot(xh, g2T[...], preferred_element_type=_f32, precision=_PREC)
    s_ref[:, H:] = _ext16(jnp.zeros((BLK, 1), _f32))
    r_ref[...] = jnp.sum(xh * ar[...], axis=1)


def _tc1(xp, weT, be, wlT, bl, a1T, g2T, ar):
    full = lambda s: pl.BlockSpec(s, lambda i: (0, 0))
    nblk = lambda s: pl.BlockSpec(s, lambda i: (i, 0))
    return pl.pallas_call(
        _tc1_body,
        grid=(NP // BLK,),
        in_specs=[nblk((BLK, H)), full((H, H)), full((1, H)), full((H, H)),
                  full((1, H)), full((H, H)), full((H, H)), full((1, H))],
        out_specs=[nblk((BLK, H)), nblk((BLK, H)), nblk((BLK, HE)),
                   pl.BlockSpec((BLK,), lambda i: (i,))],
        out_shape=[jax.ShapeDtypeStruct((NP, H), _f32),
                   jax.ShapeDtypeStruct((NP, H), _f32),
                   jax.ShapeDtypeStruct((NP, HE), _f32),
                   jax.ShapeDtypeStruct((NP,), _f32)],
    )(xp, weT, be, wlT, bl, a1T, g2T, ar)


def _tc2_body(pj_ref, ea_ref, b2T, al, t_ref):
    z = pj_ref[...] + jnp.dot(ea_ref[...], b2T[...],
                              preferred_element_type=_f32, precision=_PREC)
    m = _leaky(z)
    t_ref[...] = jnp.dot(m, al[0], preferred_element_type=_f32, precision=_PREC)


def _tc2(pj, ea, b2T, al):
    full = lambda s: pl.BlockSpec(s, lambda i: (0, 0))
    eblk = lambda s: pl.BlockSpec(s, lambda i: (i, 0))
    return pl.pallas_call(
        _tc2_body,
        grid=(EP // EB,),
        in_specs=[eblk((EB, H)), eblk((EB, ED)), full((ED, H)), full((1, H))],
        out_specs=[pl.BlockSpec((EB,), lambda i: (i,))],
        out_shape=[jax.ShapeDtypeStruct((EP,), _f32)],
    )(pj, ea, b2T, al)[0]


def _tc3_body(acc_ref, xh_ref, gcb, wihT, whhT, bih, bhh,
              aclT, asrc, adst,
              xh2_ref, xl_ref, v_ref):
    hsum = acc_ref[0, :, :H] + acc_ref[1, :, :H]
    den = acc_ref[0, :, H + 1:H + 2] + acc_ref[1, :, H + 1:H + 2]
    h = _elu(hsum / (den + 1e-16) + gcb[...])
    xh2 = jnp.maximum(
        _gru_math(h, xh_ref[...], wihT[...], whhT[...], bih[...], bhh[...]),
        0.0)
    xh2_ref[...] = xh2
    xl = jnp.dot(xh2, aclT[...], preferred_element_type=_f32, precision=_PREC)
    xl_ref[:, :H] = xl
    xl_ref[:, H:] = _ext16(jnp.sum(xl * asrc[...], axis=1, keepdims=True))
    v_ref[...] = jnp.sum(xl * adst[...], axis=1)


def _tc3(acc, xh, gcb, wihT, whhT, bih, bhh, aclT, asrc, adst):
    full = lambda s: pl.BlockSpec(s, lambda i: (0, 0))
    nblk = lambda s: pl.BlockSpec(s, lambda i: (i, 0))
    return pl.pallas_call(
        _tc3_body,
        grid=(NP // BLK,),
        in_specs=[pl.BlockSpec((NC, BLK, HE), lambda i: (0, i, 0)),
                  nblk((BLK, H)), full((1, H)),
                  full((H, 3 * H)), full((H, 3 * H)),
                  full((1, 3 * H)), full((1, 3 * H)),
                  full((H, H)), full((1, H)), full((1, H))],
        out_specs=[nblk((BLK, H)), nblk((BLK, HE)),
                   pl.BlockSpec((BLK,), lambda i: (i,))],
        out_shape=[jax.ShapeDtypeStruct((NP, H), _f32),
                   jax.ShapeDtypeStruct((NP, HE), _f32),
                   jax.ShapeDtypeStruct((NP,), _f32)],
    )(acc, xh, gcb, wihT, whhT, bih, bhh, aclT, asrc, adst)


def _tc4a_body(acc_ref, xh2_ref, acb, wihT, whhT, bih, bhh,
               mclT, msrc, bat_ref,
               xs_ref, su_ref, seg_ref):
    i = pl.program_id(0)
    ng = pl.num_programs(0)
    hsum = acc_ref[0, :, :H] + acc_ref[1, :, :H]
    den = acc_ref[0, :, H + 1:H + 2] + acc_ref[1, :, H + 1:H + 2]
    h2 = _elu(hsum / (den + 1e-16) + acb[...])
    xh3 = jnp.maximum(
        _gru_math(h2, xh2_ref[...], wihT[...], whhT[...], bih[...], bhh[...]),
        0.0)
    xs = jnp.dot(xh3, mclT[...], preferred_element_type=_f32, precision=_PREC)
    xs_ref[...] = xs
    su_ref[...] = jnp.sum(xs * msrc[...], axis=1, keepdims=True)
    gidx = lax.broadcasted_iota(jnp.int32, (BLK, G), 1)
    mask = (bat_ref[...] == gidx).astype(_f32)
    contrib = lax.dot_general(mask, xh3, (((0,), (0,)), ((), ())),
                              preferred_element_type=_f32, precision=_PREC)

    @pl.when(i == 0)
    def _():
        seg_ref[...] = contrib

    @pl.when(i > 0)
    def _():
        seg_ref[...] = seg_ref[...] + contrib

    @pl.when(i == ng - 1)
    def _():
        seg_ref[...] = jnp.maximum(seg_ref[...], 0.0)


def _tc4a(acc, xh2, acb, wihT, whhT, bih, bhh, mclT, msrc, batp):
    full = lambda s: pl.BlockSpec(s, lambda i: (0, 0))
    nblk = lambda s: pl.BlockSpec(s, lambda i: (i, 0))
    return pl.pallas_call(
        _tc4a_body,
        grid=(NP // BLK,),
        in_specs=[pl.BlockSpec((NC, BLK, HE), lambda i: (0, i, 0)),
                  nblk((BLK, H)), full((1, H)),
                  full((H, 3 * H)), full((H, 3 * H)),
                  full((1, 3 * H)), full((1, 3 * H)),
                  full((H, H)), full((1, H)), nblk((BLK, 1))],
        out_specs=[nblk((BLK, H)), nblk((BLK, 1)), full((G, H))],
        out_shape=[jax.ShapeDtypeStruct((NP, H), _f32),
                   jax.ShapeDtypeStruct((NP, 1), _f32),
                   jax.ShapeDtypeStruct((G, H), _f32)],
    )(acc, xh2, acb, wihT, whhT, bih, bhh, mclT, msrc, batp)


def _tc4b_body(out_ref, xs_ref, su_ref, bat_ref, mclT, mdst, mcb,
               wihT, whhT, bih, bhh,
               outnew_ref, num_s, den_s):
    i = pl.program_id(0)
    ng = pl.num_programs(0)
    outv = out_ref[...]
    od = jnp.dot(outv, mclT[...], preferred_element_type=_f32, precision=_PREC)
    sv = jnp.sum(od * mdst[...], axis=1, keepdims=True)
    gidx = lax.broadcasted_iota(jnp.int32, (BLK, G), 1)
    mask = (bat_ref[...] == gidx).astype(_f32)
    svn = lax.dot_general(mask, sv, (((1,), (0,)), ((), ())),
                          preferred_element_type=_f32, precision=_PREC)
    w = jnp.exp(_leaky(su_ref[...] + svn))
    nc = lax.dot_general(mask, w * xs_ref[...], (((0,), (0,)), ((), ())),
                         preferred_element_type=_f32, precision=_PREC)
    dc = lax.dot_general(mask, jnp.broadcast_to(w, (BLK, H)),
                         (((0,), (0,)), ((), ())),
                         preferred_element_type=_f32, precision=_PREC)

    @pl.when(i == 0)
    def _():
        num_s[...] = nc
        den_s[...] = dc

    @pl.when(i > 0)
    def _():
        num_s[...] = num_s[...] + nc
        den_s[...] = den_s[...] + dc

    @pl.when(i == ng - 1)
    def _():
        hm = _elu(num_s[...] / (den_s[...] + 1e-16) + mcb[...])
        outnew_ref[...] = jnp.maximum(
            _gru_math(hm, outv, wihT[...], whhT[...], bih[...], bhh[...]),
            0.0)


def _tc4b(out, xs, su, batp, mclT, mdst, mcb, wihT, whhT, bih, bhh):
    full = lambda s: pl.BlockSpec(s, lambda i: (0, 0))
    nblk = lambda s: pl.BlockSpec(s, lambda i: (i, 0))
    return pl.pallas_call(
        _tc4b_body,
        grid=(NP // BLK,),
        in_specs=[full((G, H)), nblk((BLK, H)), nblk((BLK, 1)),
                  nblk((BLK, 1)), full((H, H)), full((1, H)), full((1, H)),
                  full((H, 3 * H)), full((H, 3 * H)),
                  full((1, 3 * H)), full((1, 3 * H))],
        out_specs=[full((G, H))],
        out_shape=[jax.ShapeDtypeStruct((G, H), _f32)],
        scratch_shapes=[pltpu.VMEM((G, H), _f32), pltpu.VMEM((G, H), _f32)],
    )(out, xs, su, batp, mclT, mdst, mcb, wihT, whhT, bih, bhh)[0]


def _tc4d_body(out_ref, wl2T, bl2, wt1T, bt1, wt2T, bt2, y_ref):
    fp = jnp.dot(out_ref[...], wl2T[...], preferred_element_type=_f32, precision=_PREC) \
        + bl2[...]
    hh = jnp.maximum(
        jnp.dot(fp, wt1T[...], preferred_element_type=_f32, precision=_PREC) + bt1[...], 0.0)
    y_ref[...] = jnp.dot(hh, wt2T[...], preferred_element_type=_f32, precision=_PREC) \
        + bt2[...]


def _tc4d(out, wl2T, bl2, wt1T, bt1, wt2T, bt2):
    full = lambda s: pl.BlockSpec(s, lambda: (0, 0))
    return pl.pallas_call(
        _tc4d_body,
        in_specs=[full((G, H)), full((H, H)), full((1, H)),
                  full((H, 64)), full((1, 64)), full((64, H)), full((1, H))],
        out_specs=full((G, H)),
        out_shape=jax.ShapeDtypeStruct((G, H), _f32),
    )(out, wl2T, bl2, wt1T, bt1, wt2T, bt2)


def _sc_mesh():
    return plsc.VectorSubcoreMesh(core_axis_name="c", subcore_axis_name="s")


_SC_PARAMS = pltpu.CompilerParams(needs_layout_passes=False,
                                  use_tc_tiling_on_sc=False)
_SC_PARAMS_TILED = pltpu.CompilerParams()


def _sc_gather(table, idx):
    @functools.partial(
        pl.kernel,
        out_type=jax.ShapeDtypeStruct((EP, H), _f32),
        mesh=_sc_mesh(),
        compiler_params=_SC_PARAMS_TILED,
        scratch_types=[pltpu.VMEM((K,), jnp.int32),
                       pltpu.VMEM((K,), jnp.int32),
                       pltpu.VMEM((K, H), _f32),
                       pltpu.VMEM((K, H), _f32),
                       pltpu.SemaphoreType.DMA,
                       pltpu.SemaphoreType.DMA],
    )
    def k(tab_hbm, idx_hbm, out_hbm, idx0, idx1, rows0, rows1, g0, g1):
        cid = lax.axis_index("c")
        sid = lax.axis_index("s")
        base = (cid * NS + sid) * EPW

        pltpu.sync_copy(idx_hbm.at[pl.ds(base, K)], idx0)
        pltpu.async_copy(tab_hbm.at[idx0], rows0, g0)

        @pl.loop(0, CH - 1, step=2)
        def _(c):
            off = base + c * K
            pltpu.sync_copy(idx_hbm.at[pl.ds(off + K, K)], idx1)
            pltpu.async_copy(tab_hbm.at[idx1], rows1, g1)
            pltpu.make_async_copy(tab_hbm.at[idx0], rows0, g0).wait()
            pltpu.sync_copy(rows0, out_hbm.at[pl.ds(off, K)])

            @pl.when(c + 2 < CH)
            def _():
                pltpu.sync_copy(idx_hbm.at[pl.ds(off + 2 * K, K)], idx0)
                pltpu.async_copy(tab_hbm.at[idx0], rows0, g0)

            pltpu.make_async_copy(tab_hbm.at[idx1], rows1, g1).wait()
            pltpu.sync_copy(rows1, out_hbm.at[pl.ds(off + K, K)])

        pltpu.make_async_copy(tab_hbm.at[idx0], rows0, g0).wait()
        pltpu.sync_copy(rows0, out_hbm.at[pl.ds(base + (CH - 1) * K, K)])

    return k(table, idx)


def _sc_aggregate(table_ext, tscal, cscal, src, dst):
    HA, HB = 48, 32

    @functools.partial(
        pl.kernel,
        out_type=jax.ShapeDtypeStruct((NC, NP, HE), _f32),
        mesh=_sc_mesh(),
        compiler_params=_SC_PARAMS,
        scratch_types=[pltpu.VMEM((NP,), _f32)]
        + [pltpu.VMEM((n,), jnp.int32)
           for n in (HA, HB, HA, HB, HA, HB, HA, HB)]
        + [pltpu.VMEM((HA,), _f32), pltpu.VMEM((HB,), _f32),
           pltpu.VMEM((HA,), _f32), pltpu.VMEM((HB,), _f32),
           pltpu.VMEM((HA, HE), _f32), pltpu.VMEM((HB, HE), _f32),
           pltpu.VMEM_SHARED((NP, HE), _f32)]
        + [pltpu.SemaphoreType.DMA] * 6,
    )
    def k(tab_hbm, t_hbm, c_hbm, src_hbm, dst_hbm, acc_hbm,
          cloc, sA0, sB0, sA1, sB1, dA0, dB0, dA1, dB1,
          tA0, tB0, tA1, tB1, rowsA, rowsB, acc_sh,
          sm0, sm1, sgA, sgB, scA, scB):
        cid = lax.axis_index("c")
        sid = lax.axis_index("s")
        zv = jnp.zeros((16,), _f32)

        @pl.loop(0, HB)
        def _(i):
            for j in range(HE // 16):
                rowsB[i, pl.ds(j * 16, 16)] = zv

        @pl.loop(0, ZR // HB)
        def _(ci):
            pltpu.sync_copy(rowsB, acc_sh.at[pl.ds(sid * ZR + ci * HB, HB)])

        pltpu.sync_copy(c_hbm, cloc)
        plsc.subcore_barrier()

        base = (cid * NS + sid) * EPW
        c128 = jnp.full((16,), H, jnp.int32)
        sets = ((sA0, sB0, dA0, dB0, tA0, tB0, sm0),
                (sA1, sB1, dA1, dB1, tA1, tB1, sm1))

        def fetch(c, st):
            sa, sb, da, db, ta, tb, sem = st
            off = base + c * K
            pltpu.async_copy(src_hbm.at[pl.ds(off, HA)], sa, sem)
            pltpu.async_copy(src_hbm.at[pl.ds(off + HA, HB)], sb, sem)
            pltpu.async_copy(dst_hbm.at[pl.ds(off, HA)], da, sem)
            pltpu.async_copy(dst_hbm.at[pl.ds(off + HA, HB)], db, sem)
            pltpu.async_copy(t_hbm.at[pl.ds(off, HA)], ta, sem)
            pltpu.async_copy(t_hbm.at[pl.ds(off + HA, HB)], tb, sem)

        def fetch_wait(c, st):
            sa, sb, da, db, ta, tb, sem = st
            off = base + c * K
            pltpu.make_async_copy(src_hbm.at[pl.ds(off, HA)], sa,
                                  sem).wait()
            pltpu.make_async_copy(src_hbm.at[pl.ds(off + HA, HB)], sb,
                                  sem).wait()
            pltpu.make_async_copy(dst_hbm.at[pl.ds(off, HA)], da,
                                  sem).wait()
            pltpu.make_async_copy(dst_hbm.at[pl.ds(off + HA, HB)], db,
                                  sem).wait()
            pltpu.make_async_copy(t_hbm.at[pl.ds(off, HA)], ta, sem).wait()
            pltpu.make_async_copy(t_hbm.at[pl.ds(off + HA, HB)], tb,
                                  sem).wait()

        def compute(n, rowsX, dX, tX):
            @pl.loop(0, n // 16)
            def _(g):
                sl = pl.ds(g * 16, 16)
                ridx = lax.iota(jnp.int32, 16) + g * 16
                bv = plsc.load_gather(rowsX, [ridx, c128])
                cv = plsc.load_gather(cloc, [dX[sl]])
                gs = tX[sl] + bv + cv
                w = jnp.exp(jnp.maximum(gs, 0.01 * gs))
                for i in range(16):
                    ws = w[i]
                    for j in range(HE // 16):
                        slj = pl.ds(j * 16, 16)
                        rowsX[g * 16 + i, slj] = rowsX[g * 16 + i, slj] * ws

        def body(c, st, stn, last):
            sa, sb, da, db, ta, tb, _ = st
            pltpu.make_async_copy(tab_hbm.at[sa], rowsA, sgA).wait()
            compute(HA, rowsA, da, ta)
            pltpu.async_copy(rowsA, acc_sh.at[da], scA, add=True)
            pltpu.make_async_copy(tab_hbm.at[sb], rowsB, sgB).wait()
            if not last:
                fetch_wait(c + 1, stn)
                san, sbn = stn[0], stn[1]
                pltpu.make_async_copy(rowsA, acc_sh.at[da], scA).wait()
                pltpu.async_copy(tab_hbm.at[san], rowsA, sgA)
                compute(HB, rowsB, db, tb)
                pltpu.async_copy(rowsB, acc_sh.at[db], scB, add=True)
                pltpu.make_async_copy(rowsB, acc_sh.at[db], scB).wait()
                pltpu.async_copy(tab_hbm.at[sbn], rowsB, sgB)

                @pl.when(c + 2 < CH)
                def _():
                    fetch(c + 2, st)
            else:
                compute(HB, rowsB, db, tb)
                pltpu.async_copy(rowsB, acc_sh.at[db], scB, add=True)
                pltpu.make_async_copy(rowsA, acc_sh.at[da], scA).wait()
                pltpu.make_async_copy(rowsB, acc_sh.at[db], scB).wait()

        fetch(0, sets[0])
        fetch_wait(0, sets[0])
        pltpu.async_copy(tab_hbm.at[sA0], rowsA, sgA)
        pltpu.async_copy(tab_hbm.at[sB0], rowsB, sgB)
        fetch(1, sets[1])

        @pl.loop(0, CH - 1, step=2)
        def _(c):
            body(c, sets[0], sets[1], False)
            body(c + 1, sets[1], sets[0], False)

        body(CH - 1, sets[0], sets[1], True)

        plsc.subcore_barrier()
        pltpu.sync_copy(acc_sh.at[pl.ds(sid * ZR, ZR)],
                        acc_hbm.at[cid, pl.ds(sid * ZR, ZR)])

    return k(table_ext, tscal, cscal, src, dst)


def kernel(x, edge_index, edge_attr, batch, W_embed, b_embed, W_lin1, b_lin1,
           gc_lin1, gc_lin2, gc_att_l, gc_att_r, gc_bias,
           gru1_Wih, gru1_Whh, gru1_bih, gru1_bhh,
           ac_lin, ac_att_src, ac_att_dst, ac_bias,
           gru2_Wih, gru2_Whh, gru2_bih, gru2_bhh,
           mc_lin, mc_att_src, mc_att_dst, mc_bias,
           mgru_Wih, mgru_Whh, mgru_bih, mgru_bhh,
           W_lin2, b_lin2, W_t1, b_t1, W_t2, b_t2):
    src = edge_index[0]
    dst = edge_index[1]
    xp = jnp.pad(x, ((0, NP - N), (0, 0)))
    batp = jnp.pad(batch, (0, NP - N), constant_values=G).reshape(NP, 1)

    row = lambda b: b.reshape(1, -1)

    xh, p, s_ext, r1 = _tc1(xp, W_embed.T, row(b_embed), W_lin1.T,
                            row(b_lin1), gc_lin1[:, :H].T, gc_lin2.T,
                            row(gc_att_r))

    pj = _sc_gather(p, src)

    eap = jnp.pad(edge_attr, ((0, EP - E), (0, 0)))
    t1 = _tc2(pj, eap, gc_lin1[:, H:].T, row(gc_att_l))[:E]

    acc1 = _sc_aggregate(s_ext, t1, r1, src, dst)

    xh2, xl_ext, v = _tc3(acc1, xh, row(gc_bias),
                          gru1_Wih.T, gru1_Whh.T, row(gru1_bih),
                          row(gru1_bhh),
                          ac_lin.T, row(ac_att_src), row(ac_att_dst))

    zeros_e = jnp.zeros((E,), _f32)
    acc2 = _sc_aggregate(xl_ext, zeros_e, v, src, dst)

    xs, su, out0 = _tc4a(acc2, xh2, row(ac_bias),
                         gru2_Wih.T, gru2_Whh.T, row(gru2_bih), row(gru2_bhh),
                         mc_lin.T, row(mc_att_src), batp)

    out = out0
    for _ in range(2):
        out = _tc4b(out, xs, su, batp, mc_lin.T, row(mc_att_dst),
                    row(mc_bias), mgru_Wih.T, mgru_Whh.T, row(mgru_bih),
                    row(mgru_bhh))

    wt2T = jnp.zeros((64, H), _f32).at[:, 0].set(W_t2[0])
    bt2 = jnp.zeros((1, H), _f32).at[0, 0].set(b_t2[0])
    yfull = _tc4d(out, W_lin2.T, row(b_lin2), W_t1.T, row(b_t1), wt2T, bt2)
    return yfull[:, 0:1]
